# Initial kernel scaffold; baseline (speedup 1.0000x reference)
import jax
import jax.numpy as jnp
from jax.experimental import pallas as pl


def kernel(x, mask, params, length):
    raise NotImplementedError("write your pallas kernel here")


# speedup vs baseline: 1.1411x; 1.1411x over previous
import functools
import math

import jax
import jax.numpy as jnp
import numpy as np
from jax.experimental import pallas as pl
from jax.experimental.pallas import tpu as pltpu
from jax.experimental.pallas import tpu_sc as plsc

C_IN = 7
D_MODEL = 768
N_HEADS = 12
D_FF = 768
E_LAYERS = 2
CLASS_NUM = 10
B = 2
L = 4096
DH = 64
U = 9
G = B * N_HEADS
NW = 32
CHUNK = L // NW
RB = 512
NRB = (B * L) // RB


@functools.lru_cache(maxsize=None)
def _pos_embedding() -> np.ndarray:
    position = np.arange(L, dtype=np.float64)[:, None]
    div_term = np.exp(np.arange(0, D_MODEL, 2, dtype=np.float64)
                      * (-math.log(10000.0) / D_MODEL))
    pe = np.zeros((L, D_MODEL), dtype=np.float32)
    pe[:, 0::2] = np.sin(position * div_term).astype(np.float32)
    pe[:, 1::2] = np.cos(position * div_term).astype(np.float32)
    pos32 = np.arange(L, dtype=np.float32)[:, None]
    div32 = np.exp(np.arange(0, D_MODEL, 2, dtype=np.float32)
                   * np.float32(-math.log(10000.0) / D_MODEL)).astype(np.float32)
    pe[:, 0::2] = np.sin(pos32 * div32)
    pe[:, 1::2] = np.cos(pos32 * div32)
    return pe


def _sample_indices(layer: int):
    rng = jax.random.fold_in(jax.random.key(42), layer)
    return jax.random.randint(rng, (L, U), 0, L).astype(jnp.int32)


def _gather_indices(layer: int):
    idx = _sample_indices(layer).reshape(1, NW, U, 128)
    offs = (jnp.arange(G, dtype=jnp.int32) * L).reshape(G, 1, 1, 1)
    return idx + offs


def _ln(x, g, b, eps=1e-5):
    mu = jnp.mean(x, axis=-1, keepdims=True)
    var = jnp.mean((x - mu) ** 2, axis=-1, keepdims=True)
    return (x - mu) / jnp.sqrt(var + eps) * g + b


def _embed_body(x_ref, w_ref, pe_ref, o_ref):
    o_ref[...] = jnp.dot(x_ref[...], w_ref[...],
                         preferred_element_type=jnp.float32) + pe_ref[...]


def _embed(im, wemb):
    pe = jnp.asarray(_pos_embedding())
    return pl.pallas_call(
        _embed_body,
        grid=(NRB,),
        in_specs=[
            pl.BlockSpec((RB, C_IN * 3), lambda i: (i, 0)),
            pl.BlockSpec((C_IN * 3, D_MODEL), lambda i: (0, 0)),
            pl.BlockSpec((RB, D_MODEL), lambda i: (i % (L // RB), 0)),
        ],
        out_specs=pl.BlockSpec((RB, D_MODEL), lambda i: (i, 0)),
        out_shape=jax.ShapeDtypeStruct((B * L, D_MODEL), jnp.float32),
    )(im, wemb, pe)


def _mm_body(x_ref, w_ref, b_ref, o_ref):
    o_ref[...] = jnp.dot(x_ref[...], w_ref[...],
                         preferred_element_type=jnp.float32) + b_ref[...]


def _mm_bias(x, w, b):
    n = w.shape[1]
    return pl.pallas_call(
        _mm_body,
        grid=(NRB,),
        in_specs=[
            pl.BlockSpec((RB, D_MODEL), lambda i: (i, 0)),
            pl.BlockSpec((D_MODEL, n), lambda i: (0, 0)),
            pl.BlockSpec((1, n), lambda i: (0, 0)),
        ],
        out_specs=pl.BlockSpec((RB, n), lambda i: (i, 0)),
        out_shape=jax.ShapeDtypeStruct((x.shape[0], n), jnp.float32),
    )(x, w, b)


def _mscores(qt, kflat, idxg):
    mesh = plsc.VectorSubcoreMesh(core_axis_name="c", subcore_axis_name="s")

    @functools.partial(
        pl.kernel,
        mesh=mesh,
        compiler_params=pltpu.CompilerParams(
            use_tc_tiling_on_sc=False, needs_layout_passes=False),
        out_type=jax.ShapeDtypeStruct((G, L), jnp.float32),
        scratch_types=[
            pltpu.VMEM((DH, CHUNK), jnp.float32),
            pltpu.VMEM((U, 128), jnp.int32),
            pltpu.VMEM((CHUNK * U, DH), jnp.float32),
            pltpu.VMEM((CHUNK,), jnp.float32),
            pltpu.SemaphoreType.DMA,
        ],
    )
    def mk(qt_hbm, kflat_hbm, idx_hbm, m_hbm, qv, iv, kr, mb, sem):
        c = jax.lax.axis_index("c")
        s = jax.lax.axis_index("s")
        w = s * 2 + c

        def head_body(g, carry):
            pltpu.sync_copy(qt_hbm.at[g, w], qv)
            pltpu.sync_copy(idx_hbm.at[g, w], iv)

            def fire(t, carry2):
                pltpu.async_copy(kflat_hbm.at[iv.at[t]],
                                 kr.at[pl.ds(t * 128, 128)], sem)
                return carry2

            jax.lax.fori_loop(0, U, fire, 0)

            def drain(t, carry2):
                pltpu.make_async_copy(
                    kflat_hbm.at[iv.at[0]], kr.at[pl.ds(0, 128)], sem).wait()
                return carry2

            jax.lax.fori_loop(0, U, drain, 0)

            lane_u = jax.lax.iota(jnp.int32, 16) * U

            def grp_body(grp, carry2):
                def d_body(d, accs):
                    q16 = qv[d, pl.ds(grp * 16, 16)]
                    colv = jnp.full((16,), 0, jnp.int32) + d
                    new = []
                    for u in range(U):
                        rows = lane_u + (grp * (16 * U) + u)
                        kv = plsc.load_gather(kr, [rows, colv])
                        new.append(accs[u] + q16 * kv)
                    return tuple(new)

                accs = jax.lax.fori_loop(
                    0, DH, d_body,
                    tuple(jnp.zeros((16,), jnp.float32) for _ in range(U)))
                mx = accs[0]
                sm = accs[0]
                for u in range(1, U):
                    mx = jnp.maximum(mx, accs[u])
                    sm = sm + accs[u]
                mb[pl.ds(grp * 16, 16)] = mx - sm * (1.0 / L)
                return carry2

            jax.lax.fori_loop(0, CHUNK // 16, grp_body, 0)
            pltpu.sync_copy(mb, m_hbm.at[g, pl.ds(w * CHUNK, CHUNK)])
            return carry

        jax.lax.fori_loop(0, G, head_body, 0)

    return mk(qt, kflat, idxg)


def _topk_body(m_ref, mt_ref):
    m = m_ref[...]
    cols = jax.lax.broadcasted_iota(jnp.int32, (G, L), 1)
    picks = []
    for _ in range(U):
        mx = jnp.max(m, axis=-1, keepdims=True)
        arg = jnp.min(jnp.where(m == mx, cols, L), axis=-1, keepdims=True)
        picks.append(arg)
        m = jnp.where(cols == arg, -jnp.inf, m)
    picks += [jnp.zeros((G, 1), jnp.int32)] * (16 - U)
    mt_ref[...] = jnp.concatenate(picks, axis=1)


def _topk(m):
    return pl.pallas_call(
        _topk_body,
        out_shape=jax.ShapeDtypeStruct((G, 16), jnp.int32),
    )(m)


def _attn_body(mt_ref, q_ref, k_ref, v_ref, upd_ref, mv_ref, qr):
    for j in range(16):
        qr[pl.ds(j, 1), :] = q_ref[0, pl.ds(mt_ref[0, 0, j], 1), :]
    k = k_ref[0]
    v = v_ref[0]
    scores = jax.lax.dot_general(
        qr[...], k, (((1,), (1,)), ((), ())),
        preferred_element_type=jnp.float32) * (1.0 / math.sqrt(DH))
    mx = jnp.max(scores, axis=-1, keepdims=True)
    e = jnp.exp(scores - mx)
    attn = e / jnp.sum(e, axis=-1, keepdims=True)
    upd_ref[0] = jax.lax.dot_general(
        attn, v, (((1,), (0,)), ((), ())),
        preferred_element_type=jnp.float32)
    mv_ref[...] = jnp.mean(v, axis=0, keepdims=True)[None]


def _attn(mt, q, k, v):
    return pl.pallas_call(
        _attn_body,
        grid=(G,),
        in_specs=[
            pl.BlockSpec((1, 1, 16), lambda g: (g, 0, 0),
                         memory_space=pltpu.SMEM),
            pl.BlockSpec((1, L, DH), lambda g: (g, 0, 0)),
            pl.BlockSpec((1, L, DH), lambda g: (g, 0, 0)),
            pl.BlockSpec((1, L, DH), lambda g: (g, 0, 0)),
        ],
        out_specs=[
            pl.BlockSpec((1, 16, DH), lambda g: (g, 0, 0)),
            pl.BlockSpec((1, 1, DH), lambda g: (g, 0, 0)),
        ],
        out_shape=[
            jax.ShapeDtypeStruct((G, 16, DH), jnp.float32),
            jax.ShapeDtypeStruct((G, 1, DH), jnp.float32),
        ],
        scratch_shapes=[pltpu.VMEM((16, DH), jnp.float32)],
    )(mt.reshape(G, 1, 16), q, k, v)


def _delta_body(upd_ref, mv_ref, wo_ref, bo_ref, delta_ref, base_ref):
    g = pl.program_id(0)
    mv = mv_ref[0]
    wo = wo_ref[...]
    delta_ref[0] = jax.lax.dot_general(
        upd_ref[0] - mv, wo, (((1,), (0,)), ((), ())),
        preferred_element_type=jnp.float32)
    contrib = jax.lax.dot_general(
        mv, wo, (((1,), (0,)), ((), ())), preferred_element_type=jnp.float32)

    @pl.when(g % N_HEADS == 0)
    def _():
        base_ref[...] = (bo_ref[...] + contrib)[None]

    @pl.when(g % N_HEADS != 0)
    def _():
        base_ref[...] += contrib[None]


def _delta(upd, mv, wo, bo):
    return pl.pallas_call(
        _delta_body,
        grid=(G,),
        in_specs=[
            pl.BlockSpec((1, 16, DH), lambda g: (g, 0, 0)),
            pl.BlockSpec((1, 1, DH), lambda g: (g, 0, 0)),
            pl.BlockSpec((DH, D_MODEL), lambda g: (g % N_HEADS, 0)),
            pl.BlockSpec((1, D_MODEL), lambda g: (0, 0)),
        ],
        out_specs=[
            pl.BlockSpec((1, 16, D_MODEL), lambda g: (g, 0, 0)),
            pl.BlockSpec((1, 1, D_MODEL), lambda g: (g // N_HEADS, 0, 0)),
        ],
        out_shape=[
            jax.ShapeDtypeStruct((G, 16, D_MODEL), jnp.float32),
            jax.ShapeDtypeStruct((B, 1, D_MODEL), jnp.float32),
        ],
    )(upd, mv, wo, bo)


def _ffn_body(mt_ref, h_ref, base_ref, delta_ref, w1_ref, b1_ref,
              w2_ref, b2_ref, g1_ref, bb1_ref, g2_ref, bb2_ref, o_ref, x2):
    i = pl.program_id(0)
    x2[...] = h_ref[...] + base_ref[0]
    row0 = i * RB
    for g in range(G):
        boff = (g // N_HEADS) * L
        for j in range(U):
            r = mt_ref[g, j] + boff - row0
            rc = jnp.clip(r, 0, RB - 1)

            @pl.when((r >= 0) & (r < RB))
            def _():
                x2[pl.ds(rc, 1), :] += delta_ref[g, j, :][None, :]

    xln = _ln(x2[...], g1_ref[...], bb1_ref[...])
    y = jnp.maximum(jnp.dot(xln, w1_ref[...],
                            preferred_element_type=jnp.float32)
                    + b1_ref[...], 0.0)
    y2 = jnp.dot(y, w2_ref[...], preferred_element_type=jnp.float32) \
        + b2_ref[...]
    o_ref[...] = _ln(xln + y2, g2_ref[...], bb2_ref[...])


def _ffn(mt, h, base, delta, w1, b1, w2, b2, g1, bb1, g2, bb2):
    row1 = lambda i: (0, 0)
    return pl.pallas_call(
        _ffn_body,
        grid=(NRB,),
        in_specs=[
            pl.BlockSpec(memory_space=pltpu.SMEM),
            pl.BlockSpec((RB, D_MODEL), lambda i: (i, 0)),
            pl.BlockSpec((1, 1, D_MODEL), lambda i: (i // (L // RB), 0, 0)),
            pl.BlockSpec((G, 16, D_MODEL), lambda i: (0, 0, 0)),
            pl.BlockSpec((D_MODEL, D_FF), row1),
            pl.BlockSpec((1, D_FF), row1),
            pl.BlockSpec((D_FF, D_MODEL), row1),
            pl.BlockSpec((1, D_MODEL), row1),
            pl.BlockSpec((1, D_MODEL), row1),
            pl.BlockSpec((1, D_MODEL), row1),
            pl.BlockSpec((1, D_MODEL), row1),
            pl.BlockSpec((1, D_MODEL), row1),
        ],
        out_specs=pl.BlockSpec((RB, D_MODEL), lambda i: (i, 0)),
        out_shape=jax.ShapeDtypeStruct((B * L, D_MODEL), jnp.float32),
        scratch_shapes=[pltpu.VMEM((RB, D_MODEL), jnp.float32)],
    )(mt, h, base, delta, w1, b1, w2, b2, g1, bb1, g2, bb2)


def _final_body(h_ref, mask_ref, g_ref, b_ref, pw_ref, pb_ref, o_ref, acc):
    i = pl.program_id(0)

    @pl.when(i % (L // RB) == 0)
    def _():
        acc[...] = jnp.zeros_like(acc)

    x = _ln(h_ref[...], g_ref[...], b_ref[...])
    ge = x * 0.5 * (1.0 + jax.lax.erf(x * (1.0 / math.sqrt(2.0))))
    acc[...] += jnp.sum(ge * mask_ref[...], axis=0, keepdims=True)

    @pl.when(i % (L // RB) == (L // RB) - 1)
    def _():
        o_ref[...] = (jnp.dot(acc[...] * (1.0 / L), pw_ref[...],
                              preferred_element_type=jnp.float32)
                      + pb_ref[...])[None]


def _final(h, mask2d, g, b, pw, pb):
    return pl.pallas_call(
        _final_body,
        grid=(NRB,),
        in_specs=[
            pl.BlockSpec((RB, D_MODEL), lambda i: (i, 0)),
            pl.BlockSpec((RB, 1), lambda i: (i, 0)),
            pl.BlockSpec((1, D_MODEL), lambda i: (0, 0)),
            pl.BlockSpec((1, D_MODEL), lambda i: (0, 0)),
            pl.BlockSpec((D_MODEL, CLASS_NUM), lambda i: (0, 0)),
            pl.BlockSpec((1, CLASS_NUM), lambda i: (0, 0)),
        ],
        out_specs=pl.BlockSpec((1, 1, CLASS_NUM),
                               lambda i: (i // (L // RB), 0, 0)),
        out_shape=jax.ShapeDtypeStruct((B, 1, CLASS_NUM), jnp.float32),
        scratch_shapes=[pltpu.VMEM((1, D_MODEL), jnp.float32)],
    )(h, mask2d, g, b, pw, pb)


def _layer(h, lp, li):
    wqkv = jnp.concatenate([lp['Wq'], lp['Wk'], lp['Wv']], axis=1)
    bqkv = jnp.concatenate([lp['bq'], lp['bk'], lp['bv']])[None, :]
    qkv = _mm_bias(h, wqkv, bqkv)

    def heads(a):
        return (a.reshape(B, L, N_HEADS, DH)
                 .transpose(0, 2, 1, 3).reshape(G, L, DH))

    q = heads(qkv[:, :D_MODEL])
    k = heads(qkv[:, D_MODEL:2 * D_MODEL])
    v = heads(qkv[:, 2 * D_MODEL:])

    qt = (q.transpose(0, 2, 1).reshape(G, DH, NW, CHUNK)
           .transpose(0, 2, 1, 3))
    kflat = k.reshape(G * L, DH)
    idxg = _gather_indices(li)

    m = _mscores(qt, kflat, idxg)
    mt = _topk(m)
    upd, mv = _attn(mt, q, k, v)
    delta, base = _delta(upd, mv, lp['Wo'], lp['bo'][None, :])
    return _ffn(mt, h, base, delta,
                lp['conv1_w'].T, lp['conv1_b'][None, :],
                lp['conv2_w'].T, lp['conv2_b'][None, :],
                lp['ln1_g'][None, :], lp['ln1_b'][None, :],
                lp['ln2_g'][None, :], lp['ln2_b'][None, :])


def kernel(x, mask, params, length):
    im = jnp.stack([jnp.roll(x, 1, axis=1), x, jnp.roll(x, -1, axis=1)],
                   axis=-1).reshape(B * L, C_IN * 3)
    wemb = jnp.transpose(params['emb_conv'], (1, 2, 0)).reshape(
        C_IN * 3, D_MODEL)
    h = _embed(im, wemb)
    for li, lp in enumerate(params['layers']):
        h = _layer(h, lp, li)
    out = _final(h, mask.reshape(B * L, 1),
                 params['lnf_g'][None, :], params['lnf_b'][None, :],
                 params['proj_w'], params['proj_b'][None, :])
    return out.reshape(B, CLASS_NUM)


# speedup vs baseline: 1.3396x; 1.1739x over previous
import functools
import math

import jax
import jax.numpy as jnp
import numpy as np
from jax.experimental import pallas as pl
from jax.experimental.pallas import tpu as pltpu
from jax.experimental.pallas import tpu_sc as plsc

C_IN = 7
D_MODEL = 768
N_HEADS = 12
D_FF = 768
E_LAYERS = 2
CLASS_NUM = 10
B = 2
L = 4096
DH = 64
U = 9
G = B * N_HEADS
NW = 32
CHUNK = L // NW
RB = 512
NRB = (B * L) // RB


@functools.lru_cache(maxsize=None)
def _pos_embedding() -> np.ndarray:
    position = np.arange(L, dtype=np.float64)[:, None]
    div_term = np.exp(np.arange(0, D_MODEL, 2, dtype=np.float64)
                      * (-math.log(10000.0) / D_MODEL))
    pe = np.zeros((L, D_MODEL), dtype=np.float32)
    pe[:, 0::2] = np.sin(position * div_term).astype(np.float32)
    pe[:, 1::2] = np.cos(position * div_term).astype(np.float32)
    pos32 = np.arange(L, dtype=np.float32)[:, None]
    div32 = np.exp(np.arange(0, D_MODEL, 2, dtype=np.float32)
                   * np.float32(-math.log(10000.0) / D_MODEL)).astype(np.float32)
    pe[:, 0::2] = np.sin(pos32 * div32)
    pe[:, 1::2] = np.cos(pos32 * div32)
    return pe


def _sample_indices(layer: int):
    rng = jax.random.fold_in(jax.random.key(42), layer)
    return jax.random.randint(rng, (L, U), 0, L).astype(jnp.int32)


SQ = 64


def _gather_indices(layer: int):
    idx = _sample_indices(layer).reshape(1, NW, 2, U, SQ)
    offs = (jnp.arange(G, dtype=jnp.int32) * L).reshape(G, 1, 1, 1, 1)
    return idx + offs


def _ln(x, g, b, eps=1e-5):
    mu = jnp.mean(x, axis=-1, keepdims=True)
    var = jnp.mean((x - mu) ** 2, axis=-1, keepdims=True)
    return (x - mu) / jnp.sqrt(var + eps) * g + b


def _embed_body(x_ref, w_ref, pe_ref, o_ref):
    o_ref[...] = jnp.dot(x_ref[...], w_ref[...],
                         preferred_element_type=jnp.float32) + pe_ref[...]


def _embed(im, wemb):
    pe = jnp.asarray(_pos_embedding())
    return pl.pallas_call(
        _embed_body,
        grid=(NRB,),
        in_specs=[
            pl.BlockSpec((RB, C_IN * 3), lambda i: (i, 0)),
            pl.BlockSpec((C_IN * 3, D_MODEL), lambda i: (0, 0)),
            pl.BlockSpec((RB, D_MODEL), lambda i: (i % (L // RB), 0)),
        ],
        out_specs=pl.BlockSpec((RB, D_MODEL), lambda i: (i, 0)),
        out_shape=jax.ShapeDtypeStruct((B * L, D_MODEL), jnp.float32),
    )(im, wemb, pe)


def _mm_body(x_ref, w_ref, b_ref, o_ref):
    o_ref[...] = jnp.dot(x_ref[...].astype(w_ref.dtype), w_ref[...],
                         preferred_element_type=jnp.float32) + b_ref[...]


def _mm_bias(x, w, b):
    n = w.shape[1]
    return pl.pallas_call(
        _mm_body,
        grid=(NRB,),
        in_specs=[
            pl.BlockSpec((RB, D_MODEL), lambda i: (i, 0)),
            pl.BlockSpec((D_MODEL, n), lambda i: (0, 0)),
            pl.BlockSpec((1, n), lambda i: (0, 0)),
        ],
        out_specs=pl.BlockSpec((RB, n), lambda i: (i, 0)),
        out_shape=jax.ShapeDtypeStruct((x.shape[0], n), jnp.float32),
    )(x, w, b)


NP = G // 2


def _repack_body(q_ref, k_ref, v_ref, qt_ref, qf_ref, kf_ref, vf_ref):
    qp = q_ref[0]
    qt_ref[...] = qp.T.reshape(2, DH, L)
    qf_ref[...] = qp.reshape(L, 2, DH).transpose(1, 0, 2)
    kf_ref[...] = k_ref[0].reshape(L, 2, DH).transpose(1, 0, 2)
    vf_ref[...] = v_ref[0].reshape(L, 2, DH).transpose(1, 0, 2)


def _repack(qkv3):
    nhp = N_HEADS // 2
    hm = jax.ShapeDtypeStruct((G, L, DH), jnp.float32)
    return pl.pallas_call(
        _repack_body,
        grid=(NP,),
        in_specs=[
            pl.BlockSpec((1, L, 2 * DH), lambda p: (p // nhp, 0, p % nhp)),
            pl.BlockSpec((1, L, 2 * DH),
                         lambda p: (p // nhp, 0, nhp + p % nhp)),
            pl.BlockSpec((1, L, 2 * DH),
                         lambda p: (p // nhp, 0, 2 * nhp + p % nhp)),
        ],
        out_specs=[
            pl.BlockSpec((2, DH, L), lambda p: (p, 0, 0)),
            pl.BlockSpec((2, L, DH), lambda p: (p, 0, 0)),
            pl.BlockSpec((2, L, DH), lambda p: (p, 0, 0)),
            pl.BlockSpec((2, L, DH), lambda p: (p, 0, 0)),
        ],
        out_shape=[
            jax.ShapeDtypeStruct((G, DH, L), jnp.float32),
            hm, hm, hm,
        ],
    )(qkv3, qkv3, qkv3)


def _mscores(qt, kflat, idxg):
    mesh = plsc.VectorSubcoreMesh(core_axis_name="c", subcore_axis_name="s")

    NR = SQ * U

    @functools.partial(
        pl.kernel,
        mesh=mesh,
        compiler_params=pltpu.CompilerParams(
            use_tc_tiling_on_sc=False, needs_layout_passes=False),
        out_type=jax.ShapeDtypeStruct((G, L), jnp.float32),
        scratch_types=[
            pltpu.VMEM((2, DH, SQ), jnp.float32),
            pltpu.VMEM((2, U, SQ), jnp.int32),
            pltpu.VMEM((2, NR, DH), jnp.float32),
            pltpu.VMEM((SQ,), jnp.float32),
            pltpu.SemaphoreType.DMA,
            pltpu.SemaphoreType.DMA,
            pltpu.SemaphoreType.DMA,
            pltpu.SemaphoreType.DMA,
        ],
    )
    def mk(qt_hbm, kflat_hbm, idx_hbm, m_hbm, qv2, iv2, kr2, mb,
           qsem0, qsem1, ksem0, ksem1):
        c = jax.lax.axis_index("c")
        s = jax.lax.axis_index("s")
        w = s * 2 + c
        qsems = (qsem0, qsem1)
        ksems = (ksem0, ksem1)

        def fire(it, buf):
            g = it // 2
            st = it % 2
            pltpu.sync_copy(idx_hbm.at[g, w, st], iv2.at[buf])
            pltpu.async_copy(
                qt_hbm.at[g, :, pl.ds(w * CHUNK + st * SQ, SQ)],
                qv2.at[buf], qsems[buf])

            def f(t, carry2):
                pltpu.async_copy(kflat_hbm.at[iv2.at[buf, t]],
                                 kr2.at[buf, pl.ds(t * SQ, SQ)], ksems[buf])
                return carry2

            jax.lax.fori_loop(0, U, f, 0)

        def wait_and_compute(it, buf):
            g = it // 2
            st = it % 2
            pltpu.make_async_copy(
                qt_hbm.at[0, :, pl.ds(0, SQ)], qv2.at[buf], qsems[buf]).wait()

            def drain(t, carry2):
                pltpu.make_async_copy(
                    kflat_hbm.at[iv2.at[buf, 0]], kr2.at[buf, pl.ds(0, SQ)],
                    ksems[buf]).wait()
                return carry2

            jax.lax.fori_loop(0, U, drain, 0)

            lane_u = jax.lax.iota(jnp.int32, 16) * U

            def grp_body(grp, carry2):
                def d_body(d, accs):
                    q16 = qv2[buf, d, pl.ds(grp * 16, 16)]
                    colv = jnp.full((16,), 0, jnp.int32) + d
                    new = []
                    for u in range(U):
                        rows = lane_u + (grp * (16 * U) + u)
                        kv = plsc.load_gather(kr2.at[buf], [rows, colv])
                        new.append(accs[u] + q16 * kv)
                    return tuple(new)

                accs = jax.lax.fori_loop(
                    0, DH, d_body,
                    tuple(jnp.zeros((16,), jnp.float32) for _ in range(U)))
                mx = accs[0]
                sm = accs[0]
                for u in range(1, U):
                    mx = jnp.maximum(mx, accs[u])
                    sm = sm + accs[u]
                mb[pl.ds(grp * 16, 16)] = mx - sm * (1.0 / L)
                return carry2

            jax.lax.fori_loop(0, SQ // 16, grp_body, 0)
            pltpu.sync_copy(
                mb, m_hbm.at[g, pl.ds(w * CHUNK + st * SQ, SQ)])

        fire(0, 0)

        def stage_body(it, carry):
            @pl.when(it + 1 < 2 * G)
            def _():
                @pl.when(it % 2 == 0)
                def _():
                    fire(it + 1, 1)

                @pl.when(it % 2 == 1)
                def _():
                    fire(it + 1, 0)

            @pl.when(it % 2 == 0)
            def _():
                wait_and_compute(it, 0)

            @pl.when(it % 2 == 1)
            def _():
                wait_and_compute(it, 1)

            return carry

        jax.lax.fori_loop(0, 2 * G, stage_body, 0)

    return mk(qt, kflat, idxg)


def _topk_body(m_ref, mt_ref, rows_ref):
    m = m_ref[...]
    cols = jax.lax.broadcasted_iota(jnp.int32, (G, L), 1)
    picks = []
    for _ in range(U):
        mx = jnp.max(m, axis=-1, keepdims=True)
        arg = jnp.min(jnp.where(m == mx, cols, L), axis=-1, keepdims=True)
        picks.append(arg)
        m = jnp.where(cols == arg, -jnp.inf, m)
    picks += [jnp.zeros((G, 1), jnp.int32)] * (16 - U)
    mt = jnp.concatenate(picks, axis=1)
    mt_ref[...] = mt
    boffs = (jax.lax.broadcasted_iota(jnp.int32, (G, 16), 0) // N_HEADS) * L
    j_iota = jax.lax.broadcasted_iota(jnp.int32, (G, 16), 1)
    rows_ref[...] = jnp.where(j_iota < U, mt + boffs, -1)


def _topk(m):
    return pl.pallas_call(
        _topk_body,
        out_shape=[
            jax.ShapeDtypeStruct((G, 16), jnp.int32),
            jax.ShapeDtypeStruct((G, 16), jnp.int32),
        ],
    )(m)


def _attn_body(mt_ref, q_ref, k_ref, v_ref, upd_ref, mv_ref, qr):
    for j in range(16):
        qr[pl.ds(j, 1), :] = q_ref[0, pl.ds(mt_ref[0, 0, j], 1), :]
    k = k_ref[0]
    v = v_ref[0]
    scores = jax.lax.dot_general(
        qr[...], k, (((1,), (1,)), ((), ())),
        preferred_element_type=jnp.float32) * (1.0 / math.sqrt(DH))
    mx = jnp.max(scores, axis=-1, keepdims=True)
    e = jnp.exp(scores - mx)
    attn = e / jnp.sum(e, axis=-1, keepdims=True)
    upd_ref[0] = jax.lax.dot_general(
        attn, v, (((1,), (0,)), ((), ())),
        preferred_element_type=jnp.float32)
    mv_ref[...] = jnp.mean(v, axis=0, keepdims=True)[None]


def _attn(mt, qf, kf, vf):
    return pl.pallas_call(
        _attn_body,
        grid=(G,),
        in_specs=[
            pl.BlockSpec((1, 1, 16), lambda g: (g, 0, 0),
                         memory_space=pltpu.SMEM),
            pl.BlockSpec((1, L, DH), lambda g: (g, 0, 0)),
            pl.BlockSpec((1, L, DH), lambda g: (g, 0, 0)),
            pl.BlockSpec((1, L, DH), lambda g: (g, 0, 0)),
        ],
        out_specs=[
            pl.BlockSpec((1, 16, DH), lambda g: (g, 0, 0)),
            pl.BlockSpec((1, 1, DH), lambda g: (g, 0, 0)),
        ],
        out_shape=[
            jax.ShapeDtypeStruct((G, 16, DH), jnp.float32),
            jax.ShapeDtypeStruct((G, 1, DH), jnp.float32),
        ],
        scratch_shapes=[pltpu.VMEM((16, DH), jnp.float32)],
    )(mt.reshape(G, 1, 16), qf, kf, vf)


def _delta_body(upd_ref, mv_ref, wo_ref, bo_ref, delta_ref, base_ref):
    g = pl.program_id(0)
    mv = mv_ref[0]
    wo = wo_ref[...]
    delta_ref[0] = jax.lax.dot_general(
        upd_ref[0] - mv, wo, (((1,), (0,)), ((), ())),
        preferred_element_type=jnp.float32)
    contrib = jax.lax.dot_general(
        mv, wo, (((1,), (0,)), ((), ())), preferred_element_type=jnp.float32)

    @pl.when(g % N_HEADS == 0)
    def _():
        base_ref[...] = (bo_ref[...] + contrib)[None]

    @pl.when(g % N_HEADS != 0)
    def _():
        base_ref[...] += contrib[None]


def _delta(upd, mv, wo, bo):
    return pl.pallas_call(
        _delta_body,
        grid=(G,),
        in_specs=[
            pl.BlockSpec((1, 16, DH), lambda g: (g, 0, 0)),
            pl.BlockSpec((1, 1, DH), lambda g: (g, 0, 0)),
            pl.BlockSpec((DH, D_MODEL), lambda g: (g % N_HEADS, 0)),
            pl.BlockSpec((1, D_MODEL), lambda g: (0, 0)),
        ],
        out_specs=[
            pl.BlockSpec((1, 16, D_MODEL), lambda g: (g, 0, 0)),
            pl.BlockSpec((1, 1, D_MODEL), lambda g: (g // N_HEADS, 0, 0)),
        ],
        out_shape=[
            jax.ShapeDtypeStruct((G, 16, D_MODEL), jnp.float32),
            jax.ShapeDtypeStruct((B, 1, D_MODEL), jnp.float32),
        ],
    )(upd, mv, wo, bo)


def _ffn_body(rows_ref, h_ref, base_ref, delta_ref, w1_ref, b1_ref,
              w2_ref, b2_ref, g1_ref, bb1_ref, g2_ref, bb2_ref, o_ref):
    i = pl.program_id(0)
    row0 = i * RB
    rloc = jax.lax.broadcasted_iota(jnp.int32, (RB, G * 16), 0) + row0
    onehot = (rloc == rows_ref[...]).astype(jnp.float32)
    scat = jnp.dot(onehot, delta_ref[...],
                   preferred_element_type=jnp.float32)
    x2 = h_ref[...] + base_ref[0] + scat
    xln = _ln(x2, g1_ref[...], bb1_ref[...])
    y = jnp.maximum(jnp.dot(xln.astype(w1_ref.dtype), w1_ref[...],
                            preferred_element_type=jnp.float32)
                    + b1_ref[...], 0.0)
    y2 = jnp.dot(y.astype(w2_ref.dtype), w2_ref[...],
                 preferred_element_type=jnp.float32) + b2_ref[...]
    o_ref[...] = _ln(xln + y2, g2_ref[...], bb2_ref[...])


def _ffn(rows, h, base, delta, w1, b1, w2, b2, g1, bb1, g2, bb2):
    row1 = lambda i: (0, 0)
    return pl.pallas_call(
        _ffn_body,
        grid=(NRB,),
        in_specs=[
            pl.BlockSpec((1, G * 16), lambda i: (0, 0)),
            pl.BlockSpec((RB, D_MODEL), lambda i: (i, 0)),
            pl.BlockSpec((1, 1, D_MODEL), lambda i: (i // (L // RB), 0, 0)),
            pl.BlockSpec((G * 16, D_MODEL), lambda i: (0, 0)),
            pl.BlockSpec((D_MODEL, D_FF), row1),
            pl.BlockSpec((1, D_FF), row1),
            pl.BlockSpec((D_FF, D_MODEL), row1),
            pl.BlockSpec((1, D_MODEL), row1),
            pl.BlockSpec((1, D_MODEL), row1),
            pl.BlockSpec((1, D_MODEL), row1),
            pl.BlockSpec((1, D_MODEL), row1),
            pl.BlockSpec((1, D_MODEL), row1),
        ],
        out_specs=pl.BlockSpec((RB, D_MODEL), lambda i: (i, 0)),
        out_shape=jax.ShapeDtypeStruct((B * L, D_MODEL), jnp.float32),
    )(rows, h, base, delta, w1, b1, w2, b2, g1, bb1, g2, bb2)


def _final_body(h_ref, mask_ref, g_ref, b_ref, pw_ref, pb_ref, o_ref, acc):
    i = pl.program_id(0)

    @pl.when(i % (L // RB) == 0)
    def _():
        acc[...] = jnp.zeros_like(acc)

    x = _ln(h_ref[...], g_ref[...], b_ref[...])
    ge = x * 0.5 * (1.0 + jax.lax.erf(x * (1.0 / math.sqrt(2.0))))
    acc[...] += jnp.sum(ge * mask_ref[...], axis=0, keepdims=True)

    @pl.when(i % (L // RB) == (L // RB) - 1)
    def _():
        o_ref[...] = (jnp.dot(acc[...] * (1.0 / L), pw_ref[...],
                              preferred_element_type=jnp.float32)
                      + pb_ref[...])[None]


def _final(h, mask2d, g, b, pw, pb):
    return pl.pallas_call(
        _final_body,
        grid=(NRB,),
        in_specs=[
            pl.BlockSpec((RB, D_MODEL), lambda i: (i, 0)),
            pl.BlockSpec((RB, 1), lambda i: (i, 0)),
            pl.BlockSpec((1, D_MODEL), lambda i: (0, 0)),
            pl.BlockSpec((1, D_MODEL), lambda i: (0, 0)),
            pl.BlockSpec((D_MODEL, CLASS_NUM), lambda i: (0, 0)),
            pl.BlockSpec((1, CLASS_NUM), lambda i: (0, 0)),
        ],
        out_specs=pl.BlockSpec((1, 1, CLASS_NUM),
                               lambda i: (i // (L // RB), 0, 0)),
        out_shape=jax.ShapeDtypeStruct((B, 1, CLASS_NUM), jnp.float32),
        scratch_shapes=[pltpu.VMEM((1, D_MODEL), jnp.float32)],
    )(h, mask2d, g, b, pw, pb)


def _layer(h, lp, li):
    wqkv = jnp.concatenate([lp['Wq'], lp['Wk'], lp['Wv']],
                           axis=1).astype(jnp.bfloat16)
    bqkv = jnp.concatenate([lp['bq'], lp['bk'], lp['bv']])[None, :]
    qkv = _mm_bias(h, wqkv, bqkv)
    qkv3 = qkv.reshape(B, L, 3 * D_MODEL)
    qt, qf, kf, vf = _repack(qkv3)
    idxg = _gather_indices(li)

    m = _mscores(qt, kf.reshape(G * L, DH), idxg)
    mt, rows = _topk(m)
    upd, mv = _attn(mt, qf, kf, vf)
    delta, base = _delta(upd, mv, lp['Wo'], lp['bo'][None, :])
    return _ffn(rows.reshape(1, G * 16), h, base,
                delta.reshape(G * 16, D_MODEL),
                lp['conv1_w'].T.astype(jnp.bfloat16), lp['conv1_b'][None, :],
                lp['conv2_w'].T.astype(jnp.bfloat16), lp['conv2_b'][None, :],
                lp['ln1_g'][None, :], lp['ln1_b'][None, :],
                lp['ln2_g'][None, :], lp['ln2_b'][None, :])


def kernel(x, mask, params, length):
    im = jnp.stack([jnp.roll(x, 1, axis=1), x, jnp.roll(x, -1, axis=1)],
                   axis=-1).reshape(B * L, C_IN * 3)
    wemb = jnp.transpose(params['emb_conv'], (1, 2, 0)).reshape(
        C_IN * 3, D_MODEL)
    h = _embed(im, wemb)
    for li, lp in enumerate(params['layers']):
        h = _layer(h, lp, li)
    out = _final(h, mask.reshape(B * L, 1),
                 params['lnf_g'][None, :], params['lnf_b'][None, :],
                 params['proj_w'], params['proj_b'][None, :])
    return out.reshape(B, CLASS_NUM)


# speedup vs baseline: 1.4688x; 1.0964x over previous
import functools
import math

import jax
import jax.numpy as jnp
import numpy as np
from jax.experimental import pallas as pl
from jax.experimental.pallas import tpu as pltpu
from jax.experimental.pallas import tpu_sc as plsc

C_IN = 7
D_MODEL = 768
N_HEADS = 12
D_FF = 768
E_LAYERS = 2
CLASS_NUM = 10
B = 2
L = 4096
DH = 64
U = 9
G = B * N_HEADS
NW = 32
CHUNK = L // NW
RB = 512
NRB = (B * L) // RB


@functools.lru_cache(maxsize=None)
def _pos_embedding() -> np.ndarray:
    position = np.arange(L, dtype=np.float64)[:, None]
    div_term = np.exp(np.arange(0, D_MODEL, 2, dtype=np.float64)
                      * (-math.log(10000.0) / D_MODEL))
    pe = np.zeros((L, D_MODEL), dtype=np.float32)
    pe[:, 0::2] = np.sin(position * div_term).astype(np.float32)
    pe[:, 1::2] = np.cos(position * div_term).astype(np.float32)
    pos32 = np.arange(L, dtype=np.float32)[:, None]
    div32 = np.exp(np.arange(0, D_MODEL, 2, dtype=np.float32)
                   * np.float32(-math.log(10000.0) / D_MODEL)).astype(np.float32)
    pe[:, 0::2] = np.sin(pos32 * div32)
    pe[:, 1::2] = np.cos(pos32 * div32)
    return pe


def _sample_indices(layer: int):
    rng = jax.random.fold_in(jax.random.key(42), layer)
    return jax.random.randint(rng, (L, U), 0, L).astype(jnp.int32)


SQ = 64


def _gather_indices(layer: int):
    idx = _sample_indices(layer).reshape(1, NW, 2, U, SQ)
    offs = (jnp.arange(G, dtype=jnp.int32) * L).reshape(G, 1, 1, 1, 1)
    return idx + offs


def _ln(x, g, b, eps=1e-5):
    mu = jnp.mean(x, axis=-1, keepdims=True)
    var = jnp.mean((x - mu) ** 2, axis=-1, keepdims=True)
    return (x - mu) / jnp.sqrt(var + eps) * g + b


def _embed_body(x_ref, w_ref, pe_ref, o_ref):
    o_ref[...] = jnp.dot(x_ref[...], w_ref[...],
                         preferred_element_type=jnp.float32) + pe_ref[...]


def _embed(im, wemb):
    pe = jnp.asarray(_pos_embedding())
    return pl.pallas_call(
        _embed_body,
        grid=(NRB,),
        in_specs=[
            pl.BlockSpec((RB, C_IN * 3), lambda i: (i, 0)),
            pl.BlockSpec((C_IN * 3, D_MODEL), lambda i: (0, 0)),
            pl.BlockSpec((RB, D_MODEL), lambda i: (i % (L // RB), 0)),
        ],
        out_specs=pl.BlockSpec((RB, D_MODEL), lambda i: (i, 0)),
        out_shape=jax.ShapeDtypeStruct((B * L, D_MODEL), jnp.float32),
    )(im, wemb, pe)


def _mm_body(x_ref, w_ref, b_ref, o_ref):
    o_ref[...] = jnp.dot(x_ref[...].astype(w_ref.dtype), w_ref[...],
                         preferred_element_type=jnp.float32) + b_ref[...]


def _mm_bias(x, w, b):
    n = w.shape[1]
    return pl.pallas_call(
        _mm_body,
        grid=(NRB,),
        in_specs=[
            pl.BlockSpec((RB, D_MODEL), lambda i: (i, 0)),
            pl.BlockSpec((D_MODEL, n), lambda i: (0, 0)),
            pl.BlockSpec((1, n), lambda i: (0, 0)),
        ],
        out_specs=pl.BlockSpec((RB, n), lambda i: (i, 0)),
        out_shape=jax.ShapeDtypeStruct((x.shape[0], n), jnp.float32),
    )(x, w, b)


def _qkvr_body(x_ref, w_ref, b_ref, qt_ref, qf_ref, kf_ref, vf_ref):
    acc = jnp.dot(x_ref[...].astype(w_ref.dtype), w_ref[...],
                  preferred_element_type=jnp.float32) + b_ref[...]
    q = acc[:, :D_MODEL]
    k = acc[:, D_MODEL:2 * D_MODEL]
    v = acc[:, 2 * D_MODEL:]
    qt_ref[...] = q.T.reshape(N_HEADS, DH, RB)
    qf_ref[...] = q.reshape(RB, N_HEADS, DH).transpose(1, 0, 2)
    kf_ref[...] = k.reshape(RB, N_HEADS, DH).transpose(1, 0, 2)
    vf_ref[...] = v.reshape(RB, N_HEADS, DH).transpose(1, 0, 2)


def _qkv_repack(h, w, b):
    nlb = L // RB
    hm = jax.ShapeDtypeStruct((G, L, DH), jnp.float32)
    return pl.pallas_call(
        _qkvr_body,
        grid=(NRB,),
        in_specs=[
            pl.BlockSpec((RB, D_MODEL), lambda i: (i, 0)),
            pl.BlockSpec((D_MODEL, 3 * D_MODEL), lambda i: (0, 0)),
            pl.BlockSpec((1, 3 * D_MODEL), lambda i: (0, 0)),
        ],
        out_specs=[
            pl.BlockSpec((N_HEADS, DH, RB), lambda i: (i // nlb, 0, i % nlb)),
            pl.BlockSpec((N_HEADS, RB, DH), lambda i: (i // nlb, i % nlb, 0)),
            pl.BlockSpec((N_HEADS, RB, DH), lambda i: (i // nlb, i % nlb, 0)),
            pl.BlockSpec((N_HEADS, RB, DH), lambda i: (i // nlb, i % nlb, 0)),
        ],
        out_shape=[
            jax.ShapeDtypeStruct((G, DH, L), jnp.float32),
            hm, hm, hm,
        ],
    )(h, w, b)


def _mscores(qt, kflat, idxg):
    mesh = plsc.VectorSubcoreMesh(core_axis_name="c", subcore_axis_name="s")

    NR = SQ * U

    @functools.partial(
        pl.kernel,
        mesh=mesh,
        compiler_params=pltpu.CompilerParams(
            use_tc_tiling_on_sc=False, needs_layout_passes=False),
        out_type=jax.ShapeDtypeStruct((G, L), jnp.float32),
        scratch_types=[
            pltpu.VMEM((2, DH, SQ), jnp.float32),
            pltpu.VMEM((2, U, SQ), jnp.int32),
            pltpu.VMEM((2, NR, DH), jnp.float32),
            pltpu.VMEM((SQ,), jnp.float32),
            pltpu.SemaphoreType.DMA,
            pltpu.SemaphoreType.DMA,
            pltpu.SemaphoreType.DMA,
            pltpu.SemaphoreType.DMA,
        ],
    )
    def mk(qt_hbm, kflat_hbm, idx_hbm, m_hbm, qv2, iv2, kr2, mb,
           qsem0, qsem1, ksem0, ksem1):
        c = jax.lax.axis_index("c")
        s = jax.lax.axis_index("s")
        w = s * 2 + c
        qsems = (qsem0, qsem1)
        ksems = (ksem0, ksem1)

        def fire(it, buf):
            g = it // 2
            st = it % 2
            pltpu.sync_copy(idx_hbm.at[g, w, st], iv2.at[buf])
            pltpu.async_copy(
                qt_hbm.at[g, :, pl.ds(w * CHUNK + st * SQ, SQ)],
                qv2.at[buf], qsems[buf])

            def f(t, carry2):
                pltpu.async_copy(kflat_hbm.at[iv2.at[buf, t]],
                                 kr2.at[buf, pl.ds(t * SQ, SQ)], ksems[buf])
                return carry2

            jax.lax.fori_loop(0, U, f, 0)

        def wait_and_compute(it, buf):
            g = it // 2
            st = it % 2
            pltpu.make_async_copy(
                qt_hbm.at[0, :, pl.ds(0, SQ)], qv2.at[buf], qsems[buf]).wait()

            def drain(t, carry2):
                pltpu.make_async_copy(
                    kflat_hbm.at[iv2.at[buf, 0]], kr2.at[buf, pl.ds(0, SQ)],
                    ksems[buf]).wait()
                return carry2

            jax.lax.fori_loop(0, U, drain, 0)

            lane_u = jax.lax.iota(jnp.int32, 16) * U

            def grp_body(grp, carry2):
                rowvecs = [lane_u + (grp * (16 * U) + u) for u in range(U)]
                accs = [jnp.zeros((16,), jnp.float32) for _ in range(U)]
                for d in range(DH):
                    q16 = qv2[buf, d, pl.ds(grp * 16, 16)]
                    colv = jnp.full((16,), d, jnp.int32)
                    for u in range(U):
                        kv = plsc.load_gather(kr2.at[buf],
                                              [rowvecs[u], colv])
                        accs[u] = accs[u] + q16 * kv
                mx = accs[0]
                sm = accs[0]
                for u in range(1, U):
                    mx = jnp.maximum(mx, accs[u])
                    sm = sm + accs[u]
                mb[pl.ds(grp * 16, 16)] = mx - sm * (1.0 / L)
                return carry2

            jax.lax.fori_loop(0, SQ // 16, grp_body, 0)
            pltpu.sync_copy(
                mb, m_hbm.at[g, pl.ds(w * CHUNK + st * SQ, SQ)])

        fire(0, 0)

        def stage_body(it, carry):
            @pl.when(it + 1 < 2 * G)
            def _():
                @pl.when(it % 2 == 0)
                def _():
                    fire(it + 1, 1)

                @pl.when(it % 2 == 1)
                def _():
                    fire(it + 1, 0)

            @pl.when(it % 2 == 0)
            def _():
                wait_and_compute(it, 0)

            @pl.when(it % 2 == 1)
            def _():
                wait_and_compute(it, 1)

            return carry

        jax.lax.fori_loop(0, 2 * G, stage_body, 0)

    return mk(qt, kflat, idxg)


def _topk_body(m_ref, mt_ref, rows_ref):
    m = m_ref[...]
    cols = jax.lax.broadcasted_iota(jnp.int32, (G, L), 1)
    picks = []
    for _ in range(U):
        mx = jnp.max(m, axis=-1, keepdims=True)
        arg = jnp.min(jnp.where(m == mx, cols, L), axis=-1, keepdims=True)
        picks.append(arg)
        m = jnp.where(cols == arg, -jnp.inf, m)
    picks += [jnp.zeros((G, 1), jnp.int32)] * (16 - U)
    mt = jnp.concatenate(picks, axis=1)
    mt_ref[...] = mt
    boffs = (jax.lax.broadcasted_iota(jnp.int32, (G, 16), 0) // N_HEADS) * L
    j_iota = jax.lax.broadcasted_iota(jnp.int32, (G, 16), 1)
    rows_ref[...] = jnp.where(j_iota < U, mt + boffs, -1)


def _topk(m):
    return pl.pallas_call(
        _topk_body,
        out_shape=[
            jax.ShapeDtypeStruct((G, 16), jnp.int32),
            jax.ShapeDtypeStruct((G, 16), jnp.int32),
        ],
    )(m)


def _attn_body(mt_ref, q_ref, k_ref, v_ref, upd_ref, mv_ref, qr):
    for j in range(16):
        qr[pl.ds(j, 1), :] = q_ref[0, pl.ds(mt_ref[0, 0, j], 1), :]
    k = k_ref[0]
    v = v_ref[0]
    scores = jax.lax.dot_general(
        qr[...], k, (((1,), (1,)), ((), ())),
        preferred_element_type=jnp.float32) * (1.0 / math.sqrt(DH))
    mx = jnp.max(scores, axis=-1, keepdims=True)
    e = jnp.exp(scores - mx)
    attn = e / jnp.sum(e, axis=-1, keepdims=True)
    upd_ref[0] = jax.lax.dot_general(
        attn, v, (((1,), (0,)), ((), ())),
        preferred_element_type=jnp.float32)
    mv_ref[...] = jnp.mean(v, axis=0, keepdims=True)[None]


def _attn(mt, qf, kf, vf):
    return pl.pallas_call(
        _attn_body,
        grid=(G,),
        in_specs=[
            pl.BlockSpec((1, 1, 16), lambda g: (g, 0, 0),
                         memory_space=pltpu.SMEM),
            pl.BlockSpec((1, L, DH), lambda g: (g, 0, 0)),
            pl.BlockSpec((1, L, DH), lambda g: (g, 0, 0)),
            pl.BlockSpec((1, L, DH), lambda g: (g, 0, 0)),
        ],
        out_specs=[
            pl.BlockSpec((1, 16, DH), lambda g: (g, 0, 0)),
            pl.BlockSpec((1, 1, DH), lambda g: (g, 0, 0)),
        ],
        out_shape=[
            jax.ShapeDtypeStruct((G, 16, DH), jnp.float32),
            jax.ShapeDtypeStruct((G, 1, DH), jnp.float32),
        ],
        scratch_shapes=[pltpu.VMEM((16, DH), jnp.float32)],
    )(mt.reshape(G, 1, 16), qf, kf, vf)


def _delta_body(upd_ref, mv_ref, wo_ref, bo_ref, delta_ref, base_ref):
    g = pl.program_id(0)
    mv = mv_ref[0]
    wo = wo_ref[...]
    delta_ref[0] = jax.lax.dot_general(
        upd_ref[0] - mv, wo, (((1,), (0,)), ((), ())),
        preferred_element_type=jnp.float32)
    contrib = jax.lax.dot_general(
        mv, wo, (((1,), (0,)), ((), ())), preferred_element_type=jnp.float32)

    @pl.when(g % N_HEADS == 0)
    def _():
        base_ref[...] = (bo_ref[...] + contrib)[None]

    @pl.when(g % N_HEADS != 0)
    def _():
        base_ref[...] += contrib[None]


def _delta(upd, mv, wo, bo):
    return pl.pallas_call(
        _delta_body,
        grid=(G,),
        in_specs=[
            pl.BlockSpec((1, 16, DH), lambda g: (g, 0, 0)),
            pl.BlockSpec((1, 1, DH), lambda g: (g, 0, 0)),
            pl.BlockSpec((DH, D_MODEL), lambda g: (g % N_HEADS, 0)),
            pl.BlockSpec((1, D_MODEL), lambda g: (0, 0)),
        ],
        out_specs=[
            pl.BlockSpec((1, 16, D_MODEL), lambda g: (g, 0, 0)),
            pl.BlockSpec((1, 1, D_MODEL), lambda g: (g // N_HEADS, 0, 0)),
        ],
        out_shape=[
            jax.ShapeDtypeStruct((G, 16, D_MODEL), jnp.float32),
            jax.ShapeDtypeStruct((B, 1, D_MODEL), jnp.float32),
        ],
    )(upd, mv, wo, bo)


def _ffn_body(rows_ref, h_ref, base_ref, delta_ref, w1_ref, b1_ref,
              w2_ref, b2_ref, g1_ref, bb1_ref, g2_ref, bb2_ref, o_ref):
    i = pl.program_id(0)
    row0 = i * RB
    rloc = jax.lax.broadcasted_iota(jnp.int32, (RB, G * 16), 0) + row0
    onehot = (rloc == rows_ref[...]).astype(jnp.float32)
    scat = jnp.dot(onehot, delta_ref[...],
                   preferred_element_type=jnp.float32)
    x2 = h_ref[...] + base_ref[0] + scat
    xln = _ln(x2, g1_ref[...], bb1_ref[...])
    y = jnp.maximum(jnp.dot(xln.astype(w1_ref.dtype), w1_ref[...],
                            preferred_element_type=jnp.float32)
                    + b1_ref[...], 0.0)
    y2 = jnp.dot(y.astype(w2_ref.dtype), w2_ref[...],
                 preferred_element_type=jnp.float32) + b2_ref[...]
    o_ref[...] = _ln(xln + y2, g2_ref[...], bb2_ref[...])


def _ffn(rows, h, base, delta, w1, b1, w2, b2, g1, bb1, g2, bb2):
    row1 = lambda i: (0, 0)
    return pl.pallas_call(
        _ffn_body,
        grid=(NRB,),
        in_specs=[
            pl.BlockSpec((1, G * 16), lambda i: (0, 0)),
            pl.BlockSpec((RB, D_MODEL), lambda i: (i, 0)),
            pl.BlockSpec((1, 1, D_MODEL), lambda i: (i // (L // RB), 0, 0)),
            pl.BlockSpec((G * 16, D_MODEL), lambda i: (0, 0)),
            pl.BlockSpec((D_MODEL, D_FF), row1),
            pl.BlockSpec((1, D_FF), row1),
            pl.BlockSpec((D_FF, D_MODEL), row1),
            pl.BlockSpec((1, D_MODEL), row1),
            pl.BlockSpec((1, D_MODEL), row1),
            pl.BlockSpec((1, D_MODEL), row1),
            pl.BlockSpec((1, D_MODEL), row1),
            pl.BlockSpec((1, D_MODEL), row1),
        ],
        out_specs=pl.BlockSpec((RB, D_MODEL), lambda i: (i, 0)),
        out_shape=jax.ShapeDtypeStruct((B * L, D_MODEL), jnp.float32),
    )(rows, h, base, delta, w1, b1, w2, b2, g1, bb1, g2, bb2)


def _final_body(h_ref, mask_ref, g_ref, b_ref, pw_ref, pb_ref, o_ref, acc):
    i = pl.program_id(0)

    @pl.when(i % (L // RB) == 0)
    def _():
        acc[...] = jnp.zeros_like(acc)

    x = _ln(h_ref[...], g_ref[...], b_ref[...])
    ge = x * 0.5 * (1.0 + jax.lax.erf(x * (1.0 / math.sqrt(2.0))))
    acc[...] += jnp.sum(ge * mask_ref[...], axis=0, keepdims=True)

    @pl.when(i % (L // RB) == (L // RB) - 1)
    def _():
        o_ref[...] = (jnp.dot(acc[...] * (1.0 / L), pw_ref[...],
                              preferred_element_type=jnp.float32)
                      + pb_ref[...])[None]


def _final(h, mask2d, g, b, pw, pb):
    return pl.pallas_call(
        _final_body,
        grid=(NRB,),
        in_specs=[
            pl.BlockSpec((RB, D_MODEL), lambda i: (i, 0)),
            pl.BlockSpec((RB, 1), lambda i: (i, 0)),
            pl.BlockSpec((1, D_MODEL), lambda i: (0, 0)),
            pl.BlockSpec((1, D_MODEL), lambda i: (0, 0)),
            pl.BlockSpec((D_MODEL, CLASS_NUM), lambda i: (0, 0)),
            pl.BlockSpec((1, CLASS_NUM), lambda i: (0, 0)),
        ],
        out_specs=pl.BlockSpec((1, 1, CLASS_NUM),
                               lambda i: (i // (L // RB), 0, 0)),
        out_shape=jax.ShapeDtypeStruct((B, 1, CLASS_NUM), jnp.float32),
        scratch_shapes=[pltpu.VMEM((1, D_MODEL), jnp.float32)],
    )(h, mask2d, g, b, pw, pb)


def _layer(h, lp, li):
    wqkv = jnp.concatenate([lp['Wq'], lp['Wk'], lp['Wv']],
                           axis=1).astype(jnp.bfloat16)
    bqkv = jnp.concatenate([lp['bq'], lp['bk'], lp['bv']])[None, :]
    qt, qf, kf, vf = _qkv_repack(h, wqkv, bqkv)
    idxg = _gather_indices(li)

    m = _mscores(qt, kf.reshape(G * L, DH), idxg)
    mt, rows = _topk(m)
    upd, mv = _attn(mt, qf, kf, vf)
    delta, base = _delta(upd, mv, lp['Wo'], lp['bo'][None, :])
    return _ffn(rows.reshape(1, G * 16), h, base,
                delta.reshape(G * 16, D_MODEL),
                lp['conv1_w'].T.astype(jnp.bfloat16), lp['conv1_b'][None, :],
                lp['conv2_w'].T.astype(jnp.bfloat16), lp['conv2_b'][None, :],
                lp['ln1_g'][None, :], lp['ln1_b'][None, :],
                lp['ln2_g'][None, :], lp['ln2_b'][None, :])


def kernel(x, mask, params, length):
    im = jnp.stack([jnp.roll(x, 1, axis=1), x, jnp.roll(x, -1, axis=1)],
                   axis=-1).reshape(B * L, C_IN * 3)
    wemb = jnp.transpose(params['emb_conv'], (1, 2, 0)).reshape(
        C_IN * 3, D_MODEL)
    h = _embed(im, wemb)
    for li, lp in enumerate(params['layers']):
        h = _layer(h, lp, li)
    out = _final(h, mask.reshape(B * L, 1),
                 params['lnf_g'][None, :], params['lnf_b'][None, :],
                 params['proj_w'], params['proj_b'][None, :])
    return out.reshape(B, CLASS_NUM)


# speedup vs baseline: 2.8677x; 1.9524x over previous
import functools
import math

import jax
import jax.numpy as jnp
import numpy as np
from jax.experimental import pallas as pl
from jax.experimental.pallas import tpu as pltpu
from jax.experimental.pallas import tpu_sc as plsc

C_IN = 7
D_MODEL = 768
N_HEADS = 12
D_FF = 768
E_LAYERS = 2
CLASS_NUM = 10
B = 2
L = 4096
DH = 64
U = 9
G = B * N_HEADS
NW = 32
CHUNK = L // NW
RB = 512
NRB = (B * L) // RB


@functools.lru_cache(maxsize=None)
def _pos_embedding() -> np.ndarray:
    position = np.arange(L, dtype=np.float64)[:, None]
    div_term = np.exp(np.arange(0, D_MODEL, 2, dtype=np.float64)
                      * (-math.log(10000.0) / D_MODEL))
    pe = np.zeros((L, D_MODEL), dtype=np.float32)
    pe[:, 0::2] = np.sin(position * div_term).astype(np.float32)
    pe[:, 1::2] = np.cos(position * div_term).astype(np.float32)
    pos32 = np.arange(L, dtype=np.float32)[:, None]
    div32 = np.exp(np.arange(0, D_MODEL, 2, dtype=np.float32)
                   * np.float32(-math.log(10000.0) / D_MODEL)).astype(np.float32)
    pe[:, 0::2] = np.sin(pos32 * div32)
    pe[:, 1::2] = np.cos(pos32 * div32)
    return pe


def _sample_indices(layer: int):
    rng = jax.random.fold_in(jax.random.key(42), layer)
    return jax.random.randint(rng, (L, U), 0, L).astype(jnp.int32)


SQ = 64


def _gather_indices(layer: int):
    idx = _sample_indices(layer).reshape(1, NW, 2, U, SQ)
    offs = (jnp.arange(G, dtype=jnp.int32) * L).reshape(G, 1, 1, 1, 1)
    return idx + offs


def _ln(x, g, b, eps=1e-5):
    mu = jnp.mean(x, axis=-1, keepdims=True)
    var = jnp.mean((x - mu) ** 2, axis=-1, keepdims=True)
    return (x - mu) / jnp.sqrt(var + eps) * g + b


def _embed_body(x_ref, w_ref, pe_ref, o_ref):
    o_ref[...] = jnp.dot(x_ref[...], w_ref[...],
                         preferred_element_type=jnp.float32) + pe_ref[...]


def _embed(im, wemb):
    pe = jnp.asarray(_pos_embedding())
    return pl.pallas_call(
        _embed_body,
        grid=(NRB,),
        in_specs=[
            pl.BlockSpec((RB, C_IN * 3), lambda i: (i, 0)),
            pl.BlockSpec((C_IN * 3, D_MODEL), lambda i: (0, 0)),
            pl.BlockSpec((RB, D_MODEL), lambda i: (i % (L // RB), 0)),
        ],
        out_specs=pl.BlockSpec((RB, D_MODEL), lambda i: (i, 0)),
        out_shape=jax.ShapeDtypeStruct((B * L, D_MODEL), jnp.float32),
    )(im, wemb, pe)


def _mm_body(x_ref, w_ref, b_ref, o_ref):
    o_ref[...] = jnp.dot(x_ref[...].astype(w_ref.dtype), w_ref[...],
                         preferred_element_type=jnp.float32) + b_ref[...]


def _mm_bias(x, w, b):
    n = w.shape[1]
    return pl.pallas_call(
        _mm_body,
        grid=(NRB,),
        in_specs=[
            pl.BlockSpec((RB, D_MODEL), lambda i: (i, 0)),
            pl.BlockSpec((D_MODEL, n), lambda i: (0, 0)),
            pl.BlockSpec((1, n), lambda i: (0, 0)),
        ],
        out_specs=pl.BlockSpec((RB, n), lambda i: (i, 0)),
        out_shape=jax.ShapeDtypeStruct((x.shape[0], n), jnp.float32),
    )(x, w, b)


def _qkvr_body(x_ref, w_ref, b_ref, qf_ref, kf_ref, vf_ref):
    acc = jnp.dot(x_ref[...].astype(w_ref.dtype), w_ref[...],
                  preferred_element_type=jnp.float32) + b_ref[...]
    q = acc[:, :D_MODEL]
    k = acc[:, D_MODEL:2 * D_MODEL]
    v = acc[:, 2 * D_MODEL:]
    qf_ref[...] = q.reshape(RB, N_HEADS, DH).transpose(1, 0, 2)
    kf_ref[...] = k.reshape(RB, N_HEADS, DH).transpose(1, 0, 2)
    vf_ref[...] = v.reshape(RB, N_HEADS, DH).transpose(1, 0, 2)


def _qkv_repack(h, w, b):
    nlb = L // RB
    hm = jax.ShapeDtypeStruct((G, L, DH), jnp.float32)
    return pl.pallas_call(
        _qkvr_body,
        grid=(NRB,),
        in_specs=[
            pl.BlockSpec((RB, D_MODEL), lambda i: (i, 0)),
            pl.BlockSpec((D_MODEL, 3 * D_MODEL), lambda i: (0, 0)),
            pl.BlockSpec((1, 3 * D_MODEL), lambda i: (0, 0)),
        ],
        out_specs=[
            pl.BlockSpec((N_HEADS, RB, DH), lambda i: (i // nlb, i % nlb, 0)),
            pl.BlockSpec((N_HEADS, RB, DH), lambda i: (i // nlb, i % nlb, 0)),
            pl.BlockSpec((N_HEADS, RB, DH), lambda i: (i // nlb, i % nlb, 0)),
        ],
        out_shape=[hm, hm, hm],
    )(h, w, b)


def _mscores(qf, kb, idxg):
    mesh = plsc.VectorSubcoreMesh(core_axis_name="c", subcore_axis_name="s")

    NR = SQ * U

    @functools.partial(
        pl.kernel,
        mesh=mesh,
        compiler_params=pltpu.CompilerParams(
            use_tc_tiling_on_sc=False, needs_layout_passes=False),
        out_type=jax.ShapeDtypeStruct((G, L), jnp.float32),
        scratch_types=[
            pltpu.VMEM((2, SQ, DH), jnp.float32),
            pltpu.VMEM((2, U, SQ), jnp.int32),
            pltpu.VMEM((2, NR, DH // 2), jnp.int32),
            pltpu.VMEM((SQ,), jnp.float32),
            pltpu.SemaphoreType.DMA,
            pltpu.SemaphoreType.DMA,
            pltpu.SemaphoreType.DMA,
            pltpu.SemaphoreType.DMA,
        ],
    )
    def mk(qf_hbm, kb_hbm, idx_hbm, m_hbm, qv2, iv2, kr2, mb,
           qsem0, qsem1, ksem0, ksem1):
        c = jax.lax.axis_index("c")
        s = jax.lax.axis_index("s")
        w = s * 2 + c
        qsems = (qsem0, qsem1)
        ksems = (ksem0, ksem1)

        def fire(it, buf):
            g = it // 2
            st = it % 2
            pltpu.sync_copy(idx_hbm.at[g, w, st], iv2.at[buf])
            pltpu.async_copy(
                qf_hbm.at[g, pl.ds(w * CHUNK + st * SQ, SQ), :],
                qv2.at[buf], qsems[buf])

            def f(t, carry2):
                pltpu.async_copy(kb_hbm.at[iv2.at[buf, t]],
                                 kr2.at[buf, pl.ds(t * SQ, SQ)], ksems[buf])
                return carry2

            jax.lax.fori_loop(0, U, f, 0)

        def wait_and_compute(it, buf):
            g = it // 2
            st = it % 2
            pltpu.make_async_copy(
                qf_hbm.at[0, pl.ds(0, SQ), :], qv2.at[buf], qsems[buf]).wait()

            def drain(t, carry2):
                pltpu.make_async_copy(
                    kb_hbm.at[iv2.at[buf, 0]], kr2.at[buf, pl.ds(0, SQ)],
                    ksems[buf]).wait()
                return carry2

            jax.lax.fori_loop(0, U, drain, 0)

            himask = jnp.full((16,), -65536, jnp.int32)
            lane_iota = jax.lax.iota(jnp.int32, 16)

            def q_body(i, mvec):
                q0 = qv2[buf, i, pl.ds(0, 16)]
                q1 = qv2[buf, i, pl.ds(16, 16)]
                q2 = qv2[buf, i, pl.ds(32, 16)]
                q3 = qv2[buf, i, pl.ds(48, 16)]
                dots = []
                for u in range(U):
                    j = i * U + u
                    v0 = kr2[buf, j, pl.ds(0, 16)]
                    v1 = kr2[buf, j, pl.ds(16, 16)]
                    part = (q0 * plsc.bitcast(v0 << 16, jnp.float32)
                            + q1 * plsc.bitcast(v1 << 16, jnp.float32)
                            + q2 * plsc.bitcast(v0 & himask, jnp.float32)
                            + q3 * plsc.bitcast(v1 & himask, jnp.float32))
                    dots.append(jnp.sum(part))
                mx = dots[0]
                sm = dots[0]
                for u in range(1, U):
                    mx = jnp.maximum(mx, dots[u])
                    sm = sm + dots[u]
                mi = mx - sm * (1.0 / L)
                mvec = jnp.where(lane_iota == (i % 16), mi, mvec)

                @pl.when(i % 16 == 15)
                def _():
                    mb[pl.ds((i // 16) * 16, 16)] = mvec

                return mvec

            jax.lax.fori_loop(0, SQ, q_body,
                              jnp.zeros((16,), jnp.float32))
            pltpu.sync_copy(
                mb, m_hbm.at[g, pl.ds(w * CHUNK + st * SQ, SQ)])

        fire(0, 0)

        def stage_body(it, carry):
            @pl.when(it + 1 < 2 * G)
            def _():
                @pl.when(it % 2 == 0)
                def _():
                    fire(it + 1, 1)

                @pl.when(it % 2 == 1)
                def _():
                    fire(it + 1, 0)

            @pl.when(it % 2 == 0)
            def _():
                wait_and_compute(it, 0)

            @pl.when(it % 2 == 1)
            def _():
                wait_and_compute(it, 1)

            return carry

        jax.lax.fori_loop(0, 2 * G, stage_body, 0)

    return mk(qf, kb, idxg)


def _topk_body(m_ref, mt_ref, rows_ref):
    m = m_ref[...]
    cols = jax.lax.broadcasted_iota(jnp.int32, (G, L), 1)
    picks = []
    for _ in range(U):
        mx = jnp.max(m, axis=-1, keepdims=True)
        arg = jnp.min(jnp.where(m == mx, cols, L), axis=-1, keepdims=True)
        picks.append(arg)
        m = jnp.where(cols == arg, -jnp.inf, m)
    picks += [jnp.zeros((G, 1), jnp.int32)] * (16 - U)
    mt = jnp.concatenate(picks, axis=1)
    mt_ref[...] = mt
    boffs = (jax.lax.broadcasted_iota(jnp.int32, (G, 16), 0) // N_HEADS) * L
    j_iota = jax.lax.broadcasted_iota(jnp.int32, (G, 16), 1)
    rows_ref[...] = jnp.where(j_iota < U, mt + boffs, -1)


def _topk(m):
    return pl.pallas_call(
        _topk_body,
        out_shape=[
            jax.ShapeDtypeStruct((G, 16), jnp.int32),
            jax.ShapeDtypeStruct((G, 16), jnp.int32),
        ],
    )(m)


def _attn_body(mt_ref, q_ref, k_ref, v_ref, upd_ref, mv_ref, qr):
    for j in range(16):
        qr[pl.ds(j, 1), :] = q_ref[0, pl.ds(mt_ref[0, 0, j], 1), :]
    k = k_ref[0]
    v = v_ref[0]
    scores = jax.lax.dot_general(
        qr[...], k, (((1,), (1,)), ((), ())),
        preferred_element_type=jnp.float32) * (1.0 / math.sqrt(DH))
    mx = jnp.max(scores, axis=-1, keepdims=True)
    e = jnp.exp(scores - mx)
    attn = e / jnp.sum(e, axis=-1, keepdims=True)
    upd_ref[0] = jax.lax.dot_general(
        attn, v, (((1,), (0,)), ((), ())),
        preferred_element_type=jnp.float32)
    mv_ref[...] = jnp.mean(v, axis=0, keepdims=True)[None]


def _attn(mt, qf, kf, vf):
    return pl.pallas_call(
        _attn_body,
        grid=(G,),
        in_specs=[
            pl.BlockSpec((1, 1, 16), lambda g: (g, 0, 0),
                         memory_space=pltpu.SMEM),
            pl.BlockSpec((1, L, DH), lambda g: (g, 0, 0)),
            pl.BlockSpec((1, L, DH), lambda g: (g, 0, 0)),
            pl.BlockSpec((1, L, DH), lambda g: (g, 0, 0)),
        ],
        out_specs=[
            pl.BlockSpec((1, 16, DH), lambda g: (g, 0, 0)),
            pl.BlockSpec((1, 1, DH), lambda g: (g, 0, 0)),
        ],
        out_shape=[
            jax.ShapeDtypeStruct((G, 16, DH), jnp.float32),
            jax.ShapeDtypeStruct((G, 1, DH), jnp.float32),
        ],
        scratch_shapes=[pltpu.VMEM((16, DH), jnp.float32)],
    )(mt.reshape(G, 1, 16), qf, kf, vf)


def _delta_body(upd_ref, mv_ref, wo_ref, bo_ref, delta_ref, base_ref):
    g = pl.program_id(0)
    mv = mv_ref[0]
    wo = wo_ref[...]
    delta_ref[0] = jax.lax.dot_general(
        upd_ref[0] - mv, wo, (((1,), (0,)), ((), ())),
        preferred_element_type=jnp.float32)
    contrib = jax.lax.dot_general(
        mv, wo, (((1,), (0,)), ((), ())), preferred_element_type=jnp.float32)

    @pl.when(g % N_HEADS == 0)
    def _():
        base_ref[...] = (bo_ref[...] + contrib)[None]

    @pl.when(g % N_HEADS != 0)
    def _():
        base_ref[...] += contrib[None]


def _delta(upd, mv, wo, bo):
    return pl.pallas_call(
        _delta_body,
        grid=(G,),
        in_specs=[
            pl.BlockSpec((1, 16, DH), lambda g: (g, 0, 0)),
            pl.BlockSpec((1, 1, DH), lambda g: (g, 0, 0)),
            pl.BlockSpec((DH, D_MODEL), lambda g: (g % N_HEADS, 0)),
            pl.BlockSpec((1, D_MODEL), lambda g: (0, 0)),
        ],
        out_specs=[
            pl.BlockSpec((1, 16, D_MODEL), lambda g: (g, 0, 0)),
            pl.BlockSpec((1, 1, D_MODEL), lambda g: (g // N_HEADS, 0, 0)),
        ],
        out_shape=[
            jax.ShapeDtypeStruct((G, 16, D_MODEL), jnp.float32),
            jax.ShapeDtypeStruct((B, 1, D_MODEL), jnp.float32),
        ],
    )(upd, mv, wo, bo)


def _ffn_body(rows_ref, h_ref, base_ref, delta_ref, w1_ref, b1_ref,
              w2_ref, b2_ref, g1_ref, bb1_ref, g2_ref, bb2_ref, o_ref):
    i = pl.program_id(0)
    row0 = i * RB
    rloc = jax.lax.broadcasted_iota(jnp.int32, (RB, G * 16), 0) + row0
    onehot = (rloc == rows_ref[...]).astype(jnp.float32)
    scat = jnp.dot(onehot, delta_ref[...],
                   preferred_element_type=jnp.float32)
    x2 = h_ref[...] + base_ref[0] + scat
    xln = _ln(x2, g1_ref[...], bb1_ref[...])
    y = jnp.maximum(jnp.dot(xln.astype(w1_ref.dtype), w1_ref[...],
                            preferred_element_type=jnp.float32)
                    + b1_ref[...], 0.0)
    y2 = jnp.dot(y.astype(w2_ref.dtype), w2_ref[...],
                 preferred_element_type=jnp.float32) + b2_ref[...]
    o_ref[...] = _ln(xln + y2, g2_ref[...], bb2_ref[...])


def _ffn(rows, h, base, delta, w1, b1, w2, b2, g1, bb1, g2, bb2):
    row1 = lambda i: (0, 0)
    return pl.pallas_call(
        _ffn_body,
        grid=(NRB,),
        in_specs=[
            pl.BlockSpec((1, G * 16), lambda i: (0, 0)),
            pl.BlockSpec((RB, D_MODEL), lambda i: (i, 0)),
            pl.BlockSpec((1, 1, D_MODEL), lambda i: (i // (L // RB), 0, 0)),
            pl.BlockSpec((G * 16, D_MODEL), lambda i: (0, 0)),
            pl.BlockSpec((D_MODEL, D_FF), row1),
            pl.BlockSpec((1, D_FF), row1),
            pl.BlockSpec((D_FF, D_MODEL), row1),
            pl.BlockSpec((1, D_MODEL), row1),
            pl.BlockSpec((1, D_MODEL), row1),
            pl.BlockSpec((1, D_MODEL), row1),
            pl.BlockSpec((1, D_MODEL), row1),
            pl.BlockSpec((1, D_MODEL), row1),
        ],
        out_specs=pl.BlockSpec((RB, D_MODEL), lambda i: (i, 0)),
        out_shape=jax.ShapeDtypeStruct((B * L, D_MODEL), jnp.float32),
    )(rows, h, base, delta, w1, b1, w2, b2, g1, bb1, g2, bb2)


def _final_body(h_ref, mask_ref, g_ref, b_ref, pw_ref, pb_ref, o_ref, acc):
    i = pl.program_id(0)

    @pl.when(i % (L // RB) == 0)
    def _():
        acc[...] = jnp.zeros_like(acc)

    x = _ln(h_ref[...], g_ref[...], b_ref[...])
    ge = x * 0.5 * (1.0 + jax.lax.erf(x * (1.0 / math.sqrt(2.0))))
    acc[...] += jnp.sum(ge * mask_ref[...], axis=0, keepdims=True)

    @pl.when(i % (L // RB) == (L // RB) - 1)
    def _():
        o_ref[...] = (jnp.dot(acc[...] * (1.0 / L), pw_ref[...],
                              preferred_element_type=jnp.float32)
                      + pb_ref[...])[None]


def _final(h, mask2d, g, b, pw, pb):
    return pl.pallas_call(
        _final_body,
        grid=(NRB,),
        in_specs=[
            pl.BlockSpec((RB, D_MODEL), lambda i: (i, 0)),
            pl.BlockSpec((RB, 1), lambda i: (i, 0)),
            pl.BlockSpec((1, D_MODEL), lambda i: (0, 0)),
            pl.BlockSpec((1, D_MODEL), lambda i: (0, 0)),
            pl.BlockSpec((D_MODEL, CLASS_NUM), lambda i: (0, 0)),
            pl.BlockSpec((1, CLASS_NUM), lambda i: (0, 0)),
        ],
        out_specs=pl.BlockSpec((1, 1, CLASS_NUM),
                               lambda i: (i // (L // RB), 0, 0)),
        out_shape=jax.ShapeDtypeStruct((B, 1, CLASS_NUM), jnp.float32),
        scratch_shapes=[pltpu.VMEM((1, D_MODEL), jnp.float32)],
    )(h, mask2d, g, b, pw, pb)


def _layer(h, lp, li):
    wqkv = jnp.concatenate([lp['Wq'], lp['Wk'], lp['Wv']],
                           axis=1).astype(jnp.bfloat16)
    bqkv = jnp.concatenate([lp['bq'], lp['bk'], lp['bv']])[None, :]
    qf, kf, vf = _qkv_repack(h, wqkv, bqkv)
    idxg = _gather_indices(li)

    kb16 = kf.reshape(G * L, DH).astype(jnp.bfloat16)
    kb = jax.lax.bitcast_convert_type(
        jnp.stack([kb16[:, :DH // 2], kb16[:, DH // 2:]], axis=-1), jnp.int32)

    m = _mscores(qf, kb, idxg)
    mt, rows = _topk(m)
    upd, mv = _attn(mt, qf, kf, vf)
    delta, base = _delta(upd, mv, lp['Wo'], lp['bo'][None, :])
    return _ffn(rows.reshape(1, G * 16), h, base,
                delta.reshape(G * 16, D_MODEL),
                lp['conv1_w'].T.astype(jnp.bfloat16), lp['conv1_b'][None, :],
                lp['conv2_w'].T.astype(jnp.bfloat16), lp['conv2_b'][None, :],
                lp['ln1_g'][None, :], lp['ln1_b'][None, :],
                lp['ln2_g'][None, :], lp['ln2_b'][None, :])


def kernel(x, mask, params, length):
    im = jnp.stack([jnp.roll(x, 1, axis=1), x, jnp.roll(x, -1, axis=1)],
                   axis=-1).reshape(B * L, C_IN * 3)
    wemb = jnp.transpose(params['emb_conv'], (1, 2, 0)).reshape(
        C_IN * 3, D_MODEL)
    h = _embed(im, wemb)
    for li, lp in enumerate(params['layers']):
        h = _layer(h, lp, li)
    out = _final(h, mask.reshape(B * L, 1),
                 params['lnf_g'][None, :], params['lnf_b'][None, :],
                 params['proj_w'], params['proj_b'][None, :])
    return out.reshape(B, CLASS_NUM)


# speedup vs baseline: 3.2256x; 1.1248x over previous
import functools
import math

import jax
import jax.numpy as jnp
import numpy as np
from jax.experimental import pallas as pl
from jax.experimental.pallas import tpu as pltpu
from jax.experimental.pallas import tpu_sc as plsc

C_IN = 7
D_MODEL = 768
N_HEADS = 12
D_FF = 768
E_LAYERS = 2
CLASS_NUM = 10
B = 2
L = 4096
DH = 64
U = 9
G = B * N_HEADS
NW = 32
CHUNK = L // NW
RB = 512
NRB = (B * L) // RB


@functools.lru_cache(maxsize=None)
def _pos_embedding() -> np.ndarray:
    pe = np.zeros((L, D_MODEL), dtype=np.float32)
    pos32 = np.arange(L, dtype=np.float32)[:, None]
    div32 = np.exp(np.arange(0, D_MODEL, 2, dtype=np.float32)
                   * np.float32(-math.log(10000.0) / D_MODEL)).astype(np.float32)
    pe[:, 0::2] = np.sin(pos32 * div32)
    pe[:, 1::2] = np.cos(pos32 * div32)
    return pe


def _sample_indices(layer: int):
    rng = jax.random.fold_in(jax.random.key(42), layer)
    return jax.random.randint(rng, (L, U), 0, L).astype(jnp.int32)


SQ = 64


def _gather_indices(layer: int):
    idx = _sample_indices(layer).reshape(1, NW, 2, U, SQ)
    offs = (jnp.arange(G, dtype=jnp.int32) * L).reshape(G, 1, 1, 1, 1)
    return idx + offs


def _ln(x, g, b, eps=1e-5):
    mu = jnp.mean(x, axis=-1, keepdims=True)
    var = jnp.mean((x - mu) ** 2, axis=-1, keepdims=True)
    return (x - mu) / jnp.sqrt(var + eps) * g + b


def _embed_body(x_ref, w_ref, pe_ref, o_ref):
    o_ref[...] = jnp.dot(x_ref[...], w_ref[...],
                         preferred_element_type=jnp.float32) + pe_ref[...]


def _embed(im, wemb):
    pe = jnp.asarray(_pos_embedding())
    return pl.pallas_call(
        _embed_body,
        grid=(NRB,),
        in_specs=[
            pl.BlockSpec((RB, C_IN * 3), lambda i: (i, 0)),
            pl.BlockSpec((C_IN * 3, D_MODEL), lambda i: (0, 0)),
            pl.BlockSpec((RB, D_MODEL), lambda i: (i % (L // RB), 0)),
        ],
        out_specs=pl.BlockSpec((RB, D_MODEL), lambda i: (i, 0)),
        out_shape=jax.ShapeDtypeStruct((B * L, D_MODEL), jnp.float32),
    )(im, wemb, pe)


def _rne_bf16_bits(x):
    bits = jax.lax.bitcast_convert_type(x, jnp.int32)
    lsb = jax.lax.shift_right_logical(bits, 16) & 1
    return jax.lax.shift_right_logical(bits + 32767 + lsb, 16)


def _qkvr_body(x_ref, w_ref, b_ref, qf_ref, kf_ref, vf_ref, kb_ref):
    acc = jnp.dot(x_ref[...].astype(w_ref.dtype), w_ref[...],
                  preferred_element_type=jnp.float32) + b_ref[...]
    q = acc[:, :D_MODEL]
    k = acc[:, D_MODEL:2 * D_MODEL]
    v = acc[:, 2 * D_MODEL:]
    qf_ref[...] = q.reshape(RB, N_HEADS, DH).transpose(1, 0, 2)
    kh = k.reshape(RB, N_HEADS, DH).transpose(1, 0, 2)
    kf_ref[...] = kh
    lo = _rne_bf16_bits(kh[..., :DH // 2])
    hi = _rne_bf16_bits(kh[..., DH // 2:])
    kb_ref[...] = lo | (hi << 16)
    vf_ref[...] = v.reshape(RB, N_HEADS, DH).transpose(1, 0, 2)


def _qkv_repack(h, w, b):
    nlb = L // RB
    hm = jax.ShapeDtypeStruct((G, L, DH), jnp.float32)
    return pl.pallas_call(
        _qkvr_body,
        grid=(NRB,),
        in_specs=[
            pl.BlockSpec((RB, D_MODEL), lambda i: (i, 0)),
            pl.BlockSpec((D_MODEL, 3 * D_MODEL), lambda i: (0, 0)),
            pl.BlockSpec((1, 3 * D_MODEL), lambda i: (0, 0)),
        ],
        out_specs=[
            pl.BlockSpec((N_HEADS, RB, DH), lambda i: (i // nlb, i % nlb, 0)),
            pl.BlockSpec((N_HEADS, RB, DH), lambda i: (i // nlb, i % nlb, 0)),
            pl.BlockSpec((N_HEADS, RB, DH), lambda i: (i // nlb, i % nlb, 0)),
            pl.BlockSpec((N_HEADS, RB, DH // 2),
                         lambda i: (i // nlb, i % nlb, 0)),
        ],
        out_shape=[hm, hm, hm,
                   jax.ShapeDtypeStruct((G, L, DH // 2), jnp.int32)],
    )(h, w, b)


def _mscores(qf, kb, idxg):
    mesh = plsc.VectorSubcoreMesh(core_axis_name="c", subcore_axis_name="s")

    NR = SQ * U

    @functools.partial(
        pl.kernel,
        mesh=mesh,
        compiler_params=pltpu.CompilerParams(
            use_tc_tiling_on_sc=False, needs_layout_passes=False),
        out_type=jax.ShapeDtypeStruct((G, L), jnp.float32),
        scratch_types=[
            pltpu.VMEM((2, SQ, DH), jnp.float32),
            pltpu.VMEM((2, U, SQ), jnp.int32),
            pltpu.VMEM((2, NR, DH // 2), jnp.int32),
            pltpu.VMEM((SQ,), jnp.float32),
            pltpu.SemaphoreType.DMA,
            pltpu.SemaphoreType.DMA,
            pltpu.SemaphoreType.DMA,
            pltpu.SemaphoreType.DMA,
        ],
    )
    def mk(qf_hbm, kb_hbm, idx_hbm, m_hbm, qv2, iv2, kr2, mb,
           qsem0, qsem1, ksem0, ksem1):
        c = jax.lax.axis_index("c")
        s = jax.lax.axis_index("s")
        w = s * 2 + c
        qsems = (qsem0, qsem1)
        ksems = (ksem0, ksem1)

        def fire(it, buf):
            g = it // 2
            st = it % 2
            pltpu.sync_copy(idx_hbm.at[g, w, st], iv2.at[buf])
            pltpu.async_copy(
                qf_hbm.at[g, pl.ds(w * CHUNK + st * SQ, SQ), :],
                qv2.at[buf], qsems[buf])

            def f(t, carry2):
                pltpu.async_copy(kb_hbm.at[iv2.at[buf, t]],
                                 kr2.at[buf, pl.ds(t * SQ, SQ)], ksems[buf])
                return carry2

            jax.lax.fori_loop(0, U, f, 0)

        def wait_and_compute(it, buf):
            g = it // 2
            st = it % 2
            pltpu.make_async_copy(
                qf_hbm.at[0, pl.ds(0, SQ), :], qv2.at[buf], qsems[buf]).wait()

            def drain(t, carry2):
                pltpu.make_async_copy(
                    kb_hbm.at[iv2.at[buf, 0]], kr2.at[buf, pl.ds(0, SQ)],
                    ksems[buf]).wait()
                return carry2

            jax.lax.fori_loop(0, U, drain, 0)

            himask = jnp.full((16,), -65536, jnp.int32)
            lane_iota = jax.lax.iota(jnp.int32, 16)

            def q_body(i, mvec):
                q0 = qv2[buf, i, pl.ds(0, 16)]
                q1 = qv2[buf, i, pl.ds(16, 16)]
                q2 = qv2[buf, i, pl.ds(32, 16)]
                q3 = qv2[buf, i, pl.ds(48, 16)]
                dots = []
                for u in range(U):
                    j = i * U + u
                    v0 = kr2[buf, j, pl.ds(0, 16)]
                    v1 = kr2[buf, j, pl.ds(16, 16)]
                    part = (q0 * plsc.bitcast(v0 << 16, jnp.float32)
                            + q1 * plsc.bitcast(v1 << 16, jnp.float32)
                            + q2 * plsc.bitcast(v0 & himask, jnp.float32)
                            + q3 * plsc.bitcast(v1 & himask, jnp.float32))
                    dots.append(jnp.sum(part))
                mx = dots[0]
                sm = dots[0]
                for u in range(1, U):
                    mx = jnp.maximum(mx, dots[u])
                    sm = sm + dots[u]
                mi = mx - sm * (1.0 / L)
                mvec = jnp.where(lane_iota == (i % 16), mi, mvec)

                @pl.when(i % 16 == 15)
                def _():
                    mb[pl.ds((i // 16) * 16, 16)] = mvec

                return mvec

            jax.lax.fori_loop(0, SQ, q_body,
                              jnp.zeros((16,), jnp.float32))
            pltpu.sync_copy(
                mb, m_hbm.at[g, pl.ds(w * CHUNK + st * SQ, SQ)])

        fire(0, 0)

        def stage_body(it, carry):
            @pl.when(it + 1 < 2 * G)
            def _():
                @pl.when(it % 2 == 0)
                def _():
                    fire(it + 1, 1)

                @pl.when(it % 2 == 1)
                def _():
                    fire(it + 1, 0)

            @pl.when(it % 2 == 0)
            def _():
                wait_and_compute(it, 0)

            @pl.when(it % 2 == 1)
            def _():
                wait_and_compute(it, 1)

            return carry

        jax.lax.fori_loop(0, 2 * G, stage_body, 0)

    return mk(qf, kb, idxg)


def _attnsel_body(m_ref, qf_ref, kf_ref, vf_ref, wo_ref, bo_ref,
                  rows_ref, delta_ref, base_ref, qr):
    g = pl.program_id(0)
    m = m_ref[0]
    cols = jax.lax.broadcasted_iota(jnp.int32, (1, L), 1)
    iota16 = jax.lax.broadcasted_iota(jnp.int32, (1, 16), 1)
    rv = jnp.zeros((1, 16), jnp.int32) - 1
    boff = (g // N_HEADS) * L
    qr[...] = jnp.zeros((16, DH), jnp.float32)
    for j in range(U):
        mx = jnp.max(m)
        arg = jnp.min(jnp.where(m == mx, cols, L))
        qr[pl.ds(j, 1), :] = qf_ref[0, pl.ds(arg, 1), :]
        rv = jnp.where(iota16 == j, arg + boff, rv)
        m = jnp.where(cols == arg, -jnp.inf, m)
    rows_ref[0] = rv
    k = kf_ref[0]
    v = vf_ref[0]
    scores = jax.lax.dot_general(
        qr[...], k, (((1,), (1,)), ((), ())),
        preferred_element_type=jnp.float32) * (1.0 / math.sqrt(DH))
    smx = jnp.max(scores, axis=-1, keepdims=True)
    e = jnp.exp(scores - smx)
    attn = e / jnp.sum(e, axis=-1, keepdims=True)
    upd = jax.lax.dot_general(
        attn, v, (((1,), (0,)), ((), ())), preferred_element_type=jnp.float32)
    mv = jnp.mean(v, axis=0, keepdims=True)
    wo = wo_ref[...]
    delta_ref[0] = jax.lax.dot_general(
        upd - mv, wo, (((1,), (0,)), ((), ())),
        preferred_element_type=jnp.float32)
    contrib = jax.lax.dot_general(
        mv, wo, (((1,), (0,)), ((), ())), preferred_element_type=jnp.float32)

    @pl.when(g % N_HEADS == 0)
    def _():
        base_ref[...] = (bo_ref[...] + contrib)[None]

    @pl.when(g % N_HEADS != 0)
    def _():
        base_ref[...] += contrib[None]


def _attnsel(m3, qf, kf, vf, wo, bo):
    return pl.pallas_call(
        _attnsel_body,
        grid=(G,),
        in_specs=[
            pl.BlockSpec((1, 1, L), lambda g: (g, 0, 0)),
            pl.BlockSpec((1, L, DH), lambda g: (g, 0, 0)),
            pl.BlockSpec((1, L, DH), lambda g: (g, 0, 0)),
            pl.BlockSpec((1, L, DH), lambda g: (g, 0, 0)),
            pl.BlockSpec((DH, D_MODEL), lambda g: (g % N_HEADS, 0)),
            pl.BlockSpec((1, D_MODEL), lambda g: (0, 0)),
        ],
        out_specs=[
            pl.BlockSpec((1, 1, 16), lambda g: (g, 0, 0)),
            pl.BlockSpec((1, 16, D_MODEL), lambda g: (g, 0, 0)),
            pl.BlockSpec((1, 1, D_MODEL), lambda g: (g // N_HEADS, 0, 0)),
        ],
        out_shape=[
            jax.ShapeDtypeStruct((G, 1, 16), jnp.int32),
            jax.ShapeDtypeStruct((G, 16, D_MODEL), jnp.float32),
            jax.ShapeDtypeStruct((B, 1, D_MODEL), jnp.float32),
        ],
        scratch_shapes=[pltpu.VMEM((16, DH), jnp.float32)],
    )(m3, qf, kf, vf, wo, bo)


def _ffn_body(rows_ref, h_ref, base_ref, delta_ref, w1_ref, b1_ref,
              w2_ref, b2_ref, g1_ref, bb1_ref, g2_ref, bb2_ref, o_ref):
    i = pl.program_id(0)
    row0 = i * RB
    rloc = jax.lax.broadcasted_iota(jnp.int32, (RB, G * 16), 0) + row0
    onehot = (rloc == rows_ref[...]).astype(jnp.float32)
    scat = jnp.dot(onehot, delta_ref[...],
                   preferred_element_type=jnp.float32)
    x2 = h_ref[...] + base_ref[0] + scat
    xln = _ln(x2, g1_ref[...], bb1_ref[...])
    y = jnp.maximum(jnp.dot(xln.astype(w1_ref.dtype), w1_ref[...],
                            preferred_element_type=jnp.float32)
                    + b1_ref[...], 0.0)
    y2 = jnp.dot(y.astype(w2_ref.dtype), w2_ref[...],
                 preferred_element_type=jnp.float32) + b2_ref[...]
    o_ref[...] = _ln(xln + y2, g2_ref[...], bb2_ref[...])


def _ffn(rows, h, base, delta, w1, b1, w2, b2, g1, bb1, g2, bb2):
    row1 = lambda i: (0, 0)
    return pl.pallas_call(
        _ffn_body,
        grid=(NRB,),
        in_specs=[
            pl.BlockSpec((1, G * 16), lambda i: (0, 0)),
            pl.BlockSpec((RB, D_MODEL), lambda i: (i, 0)),
            pl.BlockSpec((1, 1, D_MODEL), lambda i: (i // (L // RB), 0, 0)),
            pl.BlockSpec((G * 16, D_MODEL), lambda i: (0, 0)),
            pl.BlockSpec((D_MODEL, D_FF), row1),
            pl.BlockSpec((1, D_FF), row1),
            pl.BlockSpec((D_FF, D_MODEL), row1),
            pl.BlockSpec((1, D_MODEL), row1),
            pl.BlockSpec((1, D_MODEL), row1),
            pl.BlockSpec((1, D_MODEL), row1),
            pl.BlockSpec((1, D_MODEL), row1),
            pl.BlockSpec((1, D_MODEL), row1),
        ],
        out_specs=pl.BlockSpec((RB, D_MODEL), lambda i: (i, 0)),
        out_shape=jax.ShapeDtypeStruct((B * L, D_MODEL), jnp.float32),
    )(rows, h, base, delta, w1, b1, w2, b2, g1, bb1, g2, bb2)


def _final_body(h_ref, mask_ref, g_ref, b_ref, pw_ref, pb_ref, o_ref, acc):
    i = pl.program_id(0)

    @pl.when(i % (L // RB) == 0)
    def _():
        acc[...] = jnp.zeros_like(acc)

    x = _ln(h_ref[...], g_ref[...], b_ref[...])
    ge = x * 0.5 * (1.0 + jax.lax.erf(x * (1.0 / math.sqrt(2.0))))
    acc[...] += jnp.sum(ge * mask_ref[...], axis=0, keepdims=True)

    @pl.when(i % (L // RB) == (L // RB) - 1)
    def _():
        o_ref[...] = (jnp.dot(acc[...] * (1.0 / L), pw_ref[...],
                              preferred_element_type=jnp.float32)
                      + pb_ref[...])[None]


def _final(h, mask2d, g, b, pw, pb):
    return pl.pallas_call(
        _final_body,
        grid=(NRB,),
        in_specs=[
            pl.BlockSpec((RB, D_MODEL), lambda i: (i, 0)),
            pl.BlockSpec((RB, 1), lambda i: (i, 0)),
            pl.BlockSpec((1, D_MODEL), lambda i: (0, 0)),
            pl.BlockSpec((1, D_MODEL), lambda i: (0, 0)),
            pl.BlockSpec((D_MODEL, CLASS_NUM), lambda i: (0, 0)),
            pl.BlockSpec((1, CLASS_NUM), lambda i: (0, 0)),
        ],
        out_specs=pl.BlockSpec((1, 1, CLASS_NUM),
                               lambda i: (i // (L // RB), 0, 0)),
        out_shape=jax.ShapeDtypeStruct((B, 1, CLASS_NUM), jnp.float32),
        scratch_shapes=[pltpu.VMEM((1, D_MODEL), jnp.float32)],
    )(h, mask2d, g, b, pw, pb)


def _layer(h, lp, li):
    wqkv = jnp.concatenate([lp['Wq'], lp['Wk'], lp['Wv']],
                           axis=1).astype(jnp.bfloat16)
    bqkv = jnp.concatenate([lp['bq'], lp['bk'], lp['bv']])[None, :]
    qf, kf, vf, kb = _qkv_repack(h, wqkv, bqkv)
    idxg = _gather_indices(li)

    m = _mscores(qf, kb.reshape(G * L, DH // 2), idxg)
    rows, delta, base = _attnsel(m.reshape(G, 1, L), qf, kf, vf,
                                 lp['Wo'], lp['bo'][None, :])
    return _ffn(rows.reshape(1, G * 16), h, base,
                delta.reshape(G * 16, D_MODEL),
                lp['conv1_w'].T.astype(jnp.bfloat16), lp['conv1_b'][None, :],
                lp['conv2_w'].T.astype(jnp.bfloat16), lp['conv2_b'][None, :],
                lp['ln1_g'][None, :], lp['ln1_b'][None, :],
                lp['ln2_g'][None, :], lp['ln2_b'][None, :])


def kernel(x, mask, params, length):
    im = jnp.stack([jnp.roll(x, 1, axis=1), x, jnp.roll(x, -1, axis=1)],
                   axis=-1).reshape(B * L, C_IN * 3)
    wemb = jnp.transpose(params['emb_conv'], (1, 2, 0)).reshape(
        C_IN * 3, D_MODEL)
    h = _embed(im, wemb)
    for li, lp in enumerate(params['layers']):
        h = _layer(h, lp, li)
    out = _final(h, mask.reshape(B * L, 1),
                 params['lnf_g'][None, :], params['lnf_b'][None, :],
                 params['proj_w'], params['proj_b'][None, :])
    return out.reshape(B, CLASS_NUM)


# speedup vs baseline: 3.3259x; 1.0311x over previous
import functools
import math

import jax
import jax.numpy as jnp
import numpy as np
from jax.experimental import pallas as pl
from jax.experimental.pallas import tpu as pltpu
from jax.experimental.pallas import tpu_sc as plsc

C_IN = 7
D_MODEL = 768
N_HEADS = 12
D_FF = 768
E_LAYERS = 2
CLASS_NUM = 10
B = 2
L = 4096
DH = 64
U = 9
G = B * N_HEADS
NW = 32
CHUNK = L // NW
RB = 512
NRB = (B * L) // RB


@functools.lru_cache(maxsize=None)
def _pos_embedding() -> np.ndarray:
    pe = np.zeros((L, D_MODEL), dtype=np.float32)
    pos32 = np.arange(L, dtype=np.float32)[:, None]
    div32 = np.exp(np.arange(0, D_MODEL, 2, dtype=np.float32)
                   * np.float32(-math.log(10000.0) / D_MODEL)).astype(np.float32)
    pe[:, 0::2] = np.sin(pos32 * div32)
    pe[:, 1::2] = np.cos(pos32 * div32)
    return pe


def _sample_indices(layer: int):
    rng = jax.random.fold_in(jax.random.key(42), layer)
    return jax.random.randint(rng, (L, U), 0, L).astype(jnp.int32)


SQ = 128


def _gather_indices(layer: int):
    idx = _sample_indices(layer).reshape(1, NW, U, SQ)
    offs = (jnp.arange(G, dtype=jnp.int32) * L).reshape(G, 1, 1, 1)
    return idx + offs


def _ln(x, g, b, eps=1e-5):
    mu = jnp.mean(x, axis=-1, keepdims=True)
    var = jnp.mean((x - mu) ** 2, axis=-1, keepdims=True)
    return (x - mu) / jnp.sqrt(var + eps) * g + b


def _embed_body(x_ref, w_ref, pe_ref, o_ref):
    o_ref[...] = jnp.dot(x_ref[...], w_ref[...],
                         preferred_element_type=jnp.float32) + pe_ref[...]


def _embed(im, wemb):
    pe = jnp.asarray(_pos_embedding())
    return pl.pallas_call(
        _embed_body,
        grid=(NRB,),
        in_specs=[
            pl.BlockSpec((RB, C_IN * 3), lambda i: (i, 0)),
            pl.BlockSpec((C_IN * 3, D_MODEL), lambda i: (0, 0)),
            pl.BlockSpec((RB, D_MODEL), lambda i: (i % (L // RB), 0)),
        ],
        out_specs=pl.BlockSpec((RB, D_MODEL), lambda i: (i, 0)),
        out_shape=jax.ShapeDtypeStruct((B * L, D_MODEL), jnp.float32),
    )(im, wemb, pe)


def _rne_bf16_bits(x):
    bits = jax.lax.bitcast_convert_type(x, jnp.int32)
    lsb = jax.lax.shift_right_logical(bits, 16) & 1
    return jax.lax.shift_right_logical(bits + 32767 + lsb, 16)


def _qkvr_body(x_ref, w_ref, b_ref, qf_ref, kf_ref, vf_ref, kb_ref):
    acc = jnp.dot(x_ref[...].astype(w_ref.dtype), w_ref[...],
                  preferred_element_type=jnp.float32) + b_ref[...]
    q = acc[:, :D_MODEL]
    k = acc[:, D_MODEL:2 * D_MODEL]
    v = acc[:, 2 * D_MODEL:]
    qf_ref[...] = q.reshape(RB, N_HEADS, DH).transpose(1, 0, 2)
    kh = k.reshape(RB, N_HEADS, DH).transpose(1, 0, 2)
    kf_ref[...] = kh
    lo = _rne_bf16_bits(kh[..., :DH // 2])
    hi = _rne_bf16_bits(kh[..., DH // 2:])
    kb_ref[...] = lo | (hi << 16)
    vf_ref[...] = v.reshape(RB, N_HEADS, DH).transpose(1, 0, 2)


def _qkv_repack(h, w, b):
    nlb = L // RB
    hm = jax.ShapeDtypeStruct((G, L, DH), jnp.float32)
    return pl.pallas_call(
        _qkvr_body,
        grid=(NRB,),
        in_specs=[
            pl.BlockSpec((RB, D_MODEL), lambda i: (i, 0)),
            pl.BlockSpec((D_MODEL, 3 * D_MODEL), lambda i: (0, 0)),
            pl.BlockSpec((1, 3 * D_MODEL), lambda i: (0, 0)),
        ],
        out_specs=[
            pl.BlockSpec((N_HEADS, RB, DH), lambda i: (i // nlb, i % nlb, 0)),
            pl.BlockSpec((N_HEADS, RB, DH), lambda i: (i // nlb, i % nlb, 0)),
            pl.BlockSpec((N_HEADS, RB, DH), lambda i: (i // nlb, i % nlb, 0)),
            pl.BlockSpec((N_HEADS, RB, DH // 2),
                         lambda i: (i // nlb, i % nlb, 0)),
        ],
        out_shape=[hm, hm, hm,
                   jax.ShapeDtypeStruct((G, L, DH // 2), jnp.int32)],
    )(h, w, b)


def _mscores(qf, kb, idxg):
    mesh = plsc.VectorSubcoreMesh(core_axis_name="c", subcore_axis_name="s")

    NR = SQ * U

    @functools.partial(
        pl.kernel,
        mesh=mesh,
        compiler_params=pltpu.CompilerParams(
            use_tc_tiling_on_sc=False, needs_layout_passes=False),
        out_type=jax.ShapeDtypeStruct((G, L), jnp.float32),
        scratch_types=[
            pltpu.VMEM((2, SQ, DH), jnp.float32),
            pltpu.VMEM((2, U, SQ), jnp.int32),
            pltpu.VMEM((2, NR, DH // 2), jnp.int32),
            pltpu.VMEM((SQ,), jnp.float32),
            pltpu.SemaphoreType.DMA,
            pltpu.SemaphoreType.DMA,
            pltpu.SemaphoreType.DMA,
            pltpu.SemaphoreType.DMA,
        ],
    )
    def mk(qf_hbm, kb_hbm, idx_hbm, m_hbm, qv2, iv2, kr2, mb,
           qsem0, qsem1, ksem0, ksem1):
        c = jax.lax.axis_index("c")
        s = jax.lax.axis_index("s")
        w = s * 2 + c
        qsems = (qsem0, qsem1)
        ksems = (ksem0, ksem1)

        def fire(it, buf):
            g = it
            pltpu.sync_copy(idx_hbm.at[g, w], iv2.at[buf])
            pltpu.async_copy(
                qf_hbm.at[g, pl.ds(w * CHUNK, SQ), :],
                qv2.at[buf], qsems[buf])

            def f(t, carry2):
                pltpu.async_copy(kb_hbm.at[iv2.at[buf, t]],
                                 kr2.at[buf, pl.ds(t * SQ, SQ)], ksems[buf])
                return carry2

            jax.lax.fori_loop(0, U, f, 0)

        def wait_and_compute(it, buf):
            g = it
            pltpu.make_async_copy(
                qf_hbm.at[0, pl.ds(0, SQ), :], qv2.at[buf], qsems[buf]).wait()

            def drain(t, carry2):
                pltpu.make_async_copy(
                    kb_hbm.at[iv2.at[buf, 0]], kr2.at[buf, pl.ds(0, SQ)],
                    ksems[buf]).wait()
                return carry2

            jax.lax.fori_loop(0, U, drain, 0)

            himask = jnp.full((16,), -65536, jnp.int32)
            lane_iota = jax.lax.iota(jnp.int32, 16)

            def q_body(i, mvec):
                q0 = qv2[buf, i, pl.ds(0, 16)]
                q1 = qv2[buf, i, pl.ds(16, 16)]
                q2 = qv2[buf, i, pl.ds(32, 16)]
                q3 = qv2[buf, i, pl.ds(48, 16)]
                dots = []
                for u in range(U):
                    j = i * U + u
                    v0 = kr2[buf, j, pl.ds(0, 16)]
                    v1 = kr2[buf, j, pl.ds(16, 16)]
                    part = (q0 * plsc.bitcast(v0 << 16, jnp.float32)
                            + q1 * plsc.bitcast(v1 << 16, jnp.float32)
                            + q2 * plsc.bitcast(v0 & himask, jnp.float32)
                            + q3 * plsc.bitcast(v1 & himask, jnp.float32))
                    dots.append(jnp.sum(part))
                mx = dots[0]
                sm = dots[0]
                for u in range(1, U):
                    mx = jnp.maximum(mx, dots[u])
                    sm = sm + dots[u]
                mi = mx - sm * (1.0 / L)
                mvec = jnp.where(lane_iota == (i % 16), mi, mvec)

                @pl.when(i % 16 == 15)
                def _():
                    mb[pl.ds((i // 16) * 16, 16)] = mvec

                return mvec

            jax.lax.fori_loop(0, SQ, q_body,
                              jnp.zeros((16,), jnp.float32))
            pltpu.sync_copy(
                mb, m_hbm.at[g, pl.ds(w * CHUNK, SQ)])

        fire(0, 0)

        def stage_body(it, carry):
            @pl.when(it + 1 < G)
            def _():
                @pl.when(it % 2 == 0)
                def _():
                    fire(it + 1, 1)

                @pl.when(it % 2 == 1)
                def _():
                    fire(it + 1, 0)

            @pl.when(it % 2 == 0)
            def _():
                wait_and_compute(it, 0)

            @pl.when(it % 2 == 1)
            def _():
                wait_and_compute(it, 1)

            return carry

        jax.lax.fori_loop(0, G, stage_body, 0)

    return mk(qf, kb, idxg)


def _attnsel_body(m_ref, qf_ref, kf_ref, vf_ref, wo_ref, bo_ref,
                  rows_ref, delta_ref, base_ref, qr):
    g = pl.program_id(0)
    m = m_ref[0]
    cols = jax.lax.broadcasted_iota(jnp.int32, (1, L), 1)
    iota16 = jax.lax.broadcasted_iota(jnp.int32, (1, 16), 1)
    rv = jnp.zeros((1, 16), jnp.int32) - 1
    boff = (g // N_HEADS) * L
    qr[...] = jnp.zeros((16, DH), jnp.float32)
    for j in range(U):
        mx = jnp.max(m)
        arg = jnp.min(jnp.where(m == mx, cols, L))
        qr[pl.ds(j, 1), :] = qf_ref[0, pl.ds(arg, 1), :]
        rv = jnp.where(iota16 == j, arg + boff, rv)
        m = jnp.where(cols == arg, -jnp.inf, m)
    rows_ref[0] = rv
    k = kf_ref[0]
    v = vf_ref[0]
    scores = jax.lax.dot_general(
        qr[...], k, (((1,), (1,)), ((), ())),
        preferred_element_type=jnp.float32) * (1.0 / math.sqrt(DH))
    smx = jnp.max(scores, axis=-1, keepdims=True)
    e = jnp.exp(scores - smx)
    attn = e / jnp.sum(e, axis=-1, keepdims=True)
    upd = jax.lax.dot_general(
        attn, v, (((1,), (0,)), ((), ())), preferred_element_type=jnp.float32)
    mv = jnp.mean(v, axis=0, keepdims=True)
    wo = wo_ref[...]
    delta_ref[0] = jax.lax.dot_general(
        upd - mv, wo, (((1,), (0,)), ((), ())),
        preferred_element_type=jnp.float32)
    contrib = jax.lax.dot_general(
        mv, wo, (((1,), (0,)), ((), ())), preferred_element_type=jnp.float32)

    @pl.when(g % N_HEADS == 0)
    def _():
        base_ref[...] = (bo_ref[...] + contrib)[None]

    @pl.when(g % N_HEADS != 0)
    def _():
        base_ref[...] += contrib[None]


def _attnsel(m3, qf, kf, vf, wo, bo):
    return pl.pallas_call(
        _attnsel_body,
        grid=(G,),
        in_specs=[
            pl.BlockSpec((1, 1, L), lambda g: (g, 0, 0)),
            pl.BlockSpec((1, L, DH), lambda g: (g, 0, 0)),
            pl.BlockSpec((1, L, DH), lambda g: (g, 0, 0)),
            pl.BlockSpec((1, L, DH), lambda g: (g, 0, 0)),
            pl.BlockSpec((DH, D_MODEL), lambda g: (g % N_HEADS, 0)),
            pl.BlockSpec((1, D_MODEL), lambda g: (0, 0)),
        ],
        out_specs=[
            pl.BlockSpec((1, 1, 16), lambda g: (g, 0, 0)),
            pl.BlockSpec((1, 16, D_MODEL), lambda g: (g, 0, 0)),
            pl.BlockSpec((1, 1, D_MODEL), lambda g: (g // N_HEADS, 0, 0)),
        ],
        out_shape=[
            jax.ShapeDtypeStruct((G, 1, 16), jnp.int32),
            jax.ShapeDtypeStruct((G, 16, D_MODEL), jnp.float32),
            jax.ShapeDtypeStruct((B, 1, D_MODEL), jnp.float32),
        ],
        scratch_shapes=[pltpu.VMEM((16, DH), jnp.float32)],
    )(m3, qf, kf, vf, wo, bo)


def _ffn_body(rows_ref, h_ref, base_ref, delta_ref, w1_ref, b1_ref,
              w2_ref, b2_ref, g1_ref, bb1_ref, g2_ref, bb2_ref, o_ref):
    i = pl.program_id(0)
    row0 = i * RB
    rloc = jax.lax.broadcasted_iota(jnp.int32, (RB, G * 16), 0) + row0
    onehot = (rloc == rows_ref[...]).astype(jnp.float32)
    scat = jnp.dot(onehot, delta_ref[...],
                   preferred_element_type=jnp.float32)
    x2 = h_ref[...] + base_ref[0] + scat
    xln = _ln(x2, g1_ref[...], bb1_ref[...])
    y = jnp.maximum(jnp.dot(xln.astype(w1_ref.dtype), w1_ref[...],
                            preferred_element_type=jnp.float32)
                    + b1_ref[...], 0.0)
    y2 = jnp.dot(y.astype(w2_ref.dtype), w2_ref[...],
                 preferred_element_type=jnp.float32) + b2_ref[...]
    o_ref[...] = _ln(xln + y2, g2_ref[...], bb2_ref[...])


def _ffn(rows, h, base, delta, w1, b1, w2, b2, g1, bb1, g2, bb2):
    row1 = lambda i: (0, 0)
    return pl.pallas_call(
        _ffn_body,
        grid=(NRB,),
        in_specs=[
            pl.BlockSpec((1, G * 16), lambda i: (0, 0)),
            pl.BlockSpec((RB, D_MODEL), lambda i: (i, 0)),
            pl.BlockSpec((1, 1, D_MODEL), lambda i: (i // (L // RB), 0, 0)),
            pl.BlockSpec((G * 16, D_MODEL), lambda i: (0, 0)),
            pl.BlockSpec((D_MODEL, D_FF), row1),
            pl.BlockSpec((1, D_FF), row1),
            pl.BlockSpec((D_FF, D_MODEL), row1),
            pl.BlockSpec((1, D_MODEL), row1),
            pl.BlockSpec((1, D_MODEL), row1),
            pl.BlockSpec((1, D_MODEL), row1),
            pl.BlockSpec((1, D_MODEL), row1),
            pl.BlockSpec((1, D_MODEL), row1),
        ],
        out_specs=pl.BlockSpec((RB, D_MODEL), lambda i: (i, 0)),
        out_shape=jax.ShapeDtypeStruct((B * L, D_MODEL), jnp.float32),
    )(rows, h, base, delta, w1, b1, w2, b2, g1, bb1, g2, bb2)


def _final_body(h_ref, mask_ref, g_ref, b_ref, pw_ref, pb_ref, o_ref, acc):
    i = pl.program_id(0)

    @pl.when(i % (L // RB) == 0)
    def _():
        acc[...] = jnp.zeros_like(acc)

    x = _ln(h_ref[...], g_ref[...], b_ref[...])
    ge = x * 0.5 * (1.0 + jax.lax.erf(x * (1.0 / math.sqrt(2.0))))
    acc[...] += jnp.sum(ge * mask_ref[...], axis=0, keepdims=True)

    @pl.when(i % (L // RB) == (L // RB) - 1)
    def _():
        o_ref[...] = (jnp.dot(acc[...] * (1.0 / L), pw_ref[...],
                              preferred_element_type=jnp.float32)
                      + pb_ref[...])[None]


def _final(h, mask2d, g, b, pw, pb):
    return pl.pallas_call(
        _final_body,
        grid=(NRB,),
        in_specs=[
            pl.BlockSpec((RB, D_MODEL), lambda i: (i, 0)),
            pl.BlockSpec((RB, 1), lambda i: (i, 0)),
            pl.BlockSpec((1, D_MODEL), lambda i: (0, 0)),
            pl.BlockSpec((1, D_MODEL), lambda i: (0, 0)),
            pl.BlockSpec((D_MODEL, CLASS_NUM), lambda i: (0, 0)),
            pl.BlockSpec((1, CLASS_NUM), lambda i: (0, 0)),
        ],
        out_specs=pl.BlockSpec((1, 1, CLASS_NUM),
                               lambda i: (i // (L // RB), 0, 0)),
        out_shape=jax.ShapeDtypeStruct((B, 1, CLASS_NUM), jnp.float32),
        scratch_shapes=[pltpu.VMEM((1, D_MODEL), jnp.float32)],
    )(h, mask2d, g, b, pw, pb)


def _layer(h, lp, li):
    wqkv = jnp.concatenate([lp['Wq'], lp['Wk'], lp['Wv']],
                           axis=1).astype(jnp.bfloat16)
    bqkv = jnp.concatenate([lp['bq'], lp['bk'], lp['bv']])[None, :]
    qf, kf, vf, kb = _qkv_repack(h, wqkv, bqkv)
    idxg = _gather_indices(li)

    m = _mscores(qf, kb.reshape(G * L, DH // 2), idxg)
    rows, delta, base = _attnsel(m.reshape(G, 1, L), qf, kf, vf,
                                 lp['Wo'], lp['bo'][None, :])
    return _ffn(rows.reshape(1, G * 16), h, base,
                delta.reshape(G * 16, D_MODEL),
                lp['conv1_w'].T.astype(jnp.bfloat16), lp['conv1_b'][None, :],
                lp['conv2_w'].T.astype(jnp.bfloat16), lp['conv2_b'][None, :],
                lp['ln1_g'][None, :], lp['ln1_b'][None, :],
                lp['ln2_g'][None, :], lp['ln2_b'][None, :])


def kernel(x, mask, params, length):
    im = jnp.stack([jnp.roll(x, 1, axis=1), x, jnp.roll(x, -1, axis=1)],
                   axis=-1).reshape(B * L, C_IN * 3)
    wemb = jnp.transpose(params['emb_conv'], (1, 2, 0)).reshape(
        C_IN * 3, D_MODEL)
    h = _embed(im, wemb)
    for li, lp in enumerate(params['layers']):
        h = _layer(h, lp, li)
    out = _final(h, mask.reshape(B * L, 1),
                 params['lnf_g'][None, :], params['lnf_b'][None, :],
                 params['proj_w'], params['proj_b'][None, :])
    return out.reshape(B, CLASS_NUM)
